# Initial kernel scaffold; baseline (speedup 1.0000x reference)
import jax
import jax.numpy as jnp
from jax.experimental import pallas as pl


def kernel(tree_fnode, tree_fmess, tree_agraph, tree_bgraph, tree_cgraph, roots, graph_fnode, graph_fmess, graph_agraph, graph_bgraph, params):
    raise NotImplementedError("write your pallas kernel here")


# speedup vs baseline: 1.3515x; 1.3515x over previous
import functools

import jax
import jax.numpy as jnp
from jax import lax
from jax.experimental import pallas as pl
from jax.experimental.pallas import tpu as pltpu
from jax.experimental.pallas import tpu_sc as plsc

_HID = 256
_NC = 2
_NS = 16
_NW = _NC * _NS


def _gather_rows(table, idx, ch=128):
    n = idx.shape[0]
    d = table.shape[1]
    unit = _NW * ch
    n_pad = ((n + unit - 1) // unit) * unit
    if n_pad != n:
        idx = jnp.pad(idx, (0, n_pad - n))
    chunks = n_pad // unit

    def body(tab_ref, idx_ref, out_ref, idx_v, rows_v, sem):
        wid = lax.axis_index("s") * _NC + lax.axis_index("c")

        def step(t, carry):
            base = (wid * chunks + t) * ch
            pltpu.sync_copy(idx_ref.at[pl.ds(base, ch)], idx_v)
            pltpu.async_copy(tab_ref.at[idx_v], rows_v, sem).wait()
            pltpu.sync_copy(rows_v, out_ref.at[pl.ds(base, ch)])
            return carry

        lax.fori_loop(0, chunks, step, 0)

    mesh = plsc.VectorSubcoreMesh(core_axis_name="c", subcore_axis_name="s")
    out = pl.kernel(
        body,
        mesh=mesh,
        out_type=jax.ShapeDtypeStruct((n_pad, d), jnp.float32),
        scratch_types=[
            pltpu.VMEM((ch,), jnp.int32),
            pltpu.VMEM((ch, d), jnp.float32),
            pltpu.SemaphoreType.DMA,
        ],
    )(table, idx)
    return out[:n]


def _apply_act(y, act):
    if act == "relu":
        return jnp.maximum(y, 0.0)
    if act == "tanh":
        return jnp.tanh(y)
    return y


def _mask0(y, i_blk, be):
    row = i_blk * be + lax.broadcasted_iota(jnp.int32, (be, 1), 0)
    return jnp.where(row == 0, 0.0, y)


def _dense(x, w, b, act, mask0, be):
    n, k = x.shape
    m = w.shape[1]
    b2 = b.reshape(1, m)

    def body(x_ref, w_ref, b_ref, o_ref):
        y = jnp.dot(x_ref[...], w_ref[...], preferred_element_type=jnp.float32)
        y = y + b_ref[...]
        y = _apply_act(y, act)
        if mask0:
            y = _mask0(y, pl.program_id(0), be)
        o_ref[...] = y

    return pl.pallas_call(
        body,
        grid=(n // be,),
        in_specs=[
            pl.BlockSpec((be, k), lambda i: (i, 0)),
            pl.BlockSpec((k, m), lambda i: (0, 0)),
            pl.BlockSpec((1, m), lambda i: (0, 0)),
        ],
        out_specs=pl.BlockSpec((be, m), lambda i: (i, 0)),
        out_shape=jax.ShapeDtypeStruct((n, m), jnp.float32),
    )(x, w, b2)


def _lstm_init(a, be):
    e = a.shape[0]
    h = _HID

    def body(a_ref, h_ref, c_ref):
        av = a_ref[...]
        gi = jax.nn.sigmoid(av[:, 0 * h:1 * h])
        go = jax.nn.sigmoid(av[:, 1 * h:2 * h])
        gg = jnp.tanh(av[:, 3 * h:4 * h])
        c_new = gi * gg
        h_new = go * jnp.tanh(c_new)
        h_ref[...] = _mask0(h_new, pl.program_id(0), be)
        c_ref[...] = _mask0(c_new, pl.program_id(0), be)

    return pl.pallas_call(
        body,
        grid=(e // be,),
        in_specs=[pl.BlockSpec((be, 4 * h), lambda i: (i, 0))],
        out_specs=[
            pl.BlockSpec((be, h), lambda i: (i, 0)),
            pl.BlockSpec((be, h), lambda i: (i, 0)),
        ],
        out_shape=[
            jax.ShapeDtypeStruct((e, h), jnp.float32),
            jax.ShapeDtypeStruct((e, h), jnp.float32),
        ],
    )(a)


def _lstm_step(a, gh, gc, wh_all, wf_h, nk, be):
    e = a.shape[0]
    h = _HID

    def body(a_ref, gh_ref, gc_ref, wh_ref, wf_ref, h_ref, c_ref):
        av = a_ref[...]
        ghv = gh_ref[...]
        gcv = gc_ref[...].reshape(be, nk, h)
        hs = ghv.reshape(be, nk, h).sum(axis=1)
        zh = jnp.dot(hs, wh_ref[...], preferred_element_type=jnp.float32)
        gi = jax.nn.sigmoid(av[:, 0 * h:1 * h] + zh[:, 0 * h:1 * h])
        go = jax.nn.sigmoid(av[:, 1 * h:2 * h] + zh[:, 1 * h:2 * h])
        gg = jnp.tanh(av[:, 3 * h:4 * h] + zh[:, 2 * h:3 * h])
        fpre = jnp.dot(ghv, wf_ref[...], preferred_element_type=jnp.float32)
        gf = jax.nn.sigmoid(fpre.reshape(be, nk, h) + av[:, 2 * h:3 * h][:, None, :])
        c_new = gi * gg + (gf * gcv).sum(axis=1)
        h_new = go * jnp.tanh(c_new)
        h_ref[...] = _mask0(h_new, pl.program_id(0), be)
        c_ref[...] = _mask0(c_new, pl.program_id(0), be)

    return pl.pallas_call(
        body,
        grid=(e // be,),
        in_specs=[
            pl.BlockSpec((be, 4 * h), lambda i: (i, 0)),
            pl.BlockSpec((be * nk, h), lambda i: (i, 0)),
            pl.BlockSpec((be * nk, h), lambda i: (i, 0)),
            pl.BlockSpec((h, 3 * h), lambda i: (0, 0)),
            pl.BlockSpec((h, h), lambda i: (0, 0)),
        ],
        out_specs=[
            pl.BlockSpec((be, h), lambda i: (i, 0)),
            pl.BlockSpec((be, h), lambda i: (i, 0)),
        ],
        out_shape=[
            jax.ShapeDtypeStruct((e, h), jnp.float32),
            jax.ShapeDtypeStruct((e, h), jnp.float32),
        ],
    )(a, gh, gc, wh_all, wf_h)


def _sumcat_dense(x1, g, w, b, nk, act, mask0, be):
    n, d1 = x1.shape
    h = _HID
    m = w.shape[1]
    b2 = b.reshape(1, m)

    def body(x_ref, g_ref, w_ref, b_ref, o_ref):
        nei = g_ref[...].reshape(be, nk, h).sum(axis=1)
        xcat = jnp.concatenate([x_ref[...], nei], axis=1)
        y = jnp.dot(xcat, w_ref[...], preferred_element_type=jnp.float32)
        y = y + b_ref[...]
        y = _apply_act(y, act)
        if mask0:
            y = _mask0(y, pl.program_id(0), be)
        o_ref[...] = y

    return pl.pallas_call(
        body,
        grid=(n // be,),
        in_specs=[
            pl.BlockSpec((be, d1), lambda i: (i, 0)),
            pl.BlockSpec((be * nk, h), lambda i: (i, 0)),
            pl.BlockSpec((d1 + h, m), lambda i: (0, 0)),
            pl.BlockSpec((1, m), lambda i: (0, 0)),
        ],
        out_specs=pl.BlockSpec((be, m), lambda i: (i, 0)),
        out_shape=jax.ShapeDtypeStruct((n, m), jnp.float32),
    )(x1, g, w, b2)


def _pack_lstm(lstm, in_sz, in_pad):
    wi, wo, wf, wg = lstm["Wi"], lstm["Wog"], lstm["Wf"], lstm["W"]
    w_all = jnp.concatenate(
        [wi["w"][:in_sz], wo["w"][:in_sz], wf["w"][:in_sz], wg["w"][:in_sz]], axis=1)
    if in_pad != in_sz:
        w_all = jnp.pad(w_all, ((0, in_pad - in_sz), (0, 0)))
    b_all = jnp.concatenate([wi["b"], wo["b"], wf["b"], wg["b"]])
    wh_all = jnp.concatenate(
        [wi["w"][in_sz:], wo["w"][in_sz:], wg["w"][in_sz:]], axis=1)
    wf_h = wf["w"][in_sz:]
    return w_all, b_all, wh_all, wf_h


def _run_lstm(a, bgraph_flat, wh_all, wf_h, depth, nk, be):
    h, c = _lstm_init(a, be)
    for _ in range(depth - 1):
        gh = _gather_rows(h, bgraph_flat)
        gc = _gather_rows(c, bgraph_flat)
        h, c = _lstm_step(a, gh, gc, wh_all, wf_h, nk, be)
    return h, c


def _tree_encoder(enc, hx, f0, pos1h, agraph_flat, bgraph_flat, depth, nk, be_e, be_n):
    in_sz = _HID + pos1h.shape[1]
    in_pad = 384
    w_all, b_all, wh_all, wf_h = _pack_lstm(enc["lstm"], in_sz, in_pad)
    hx_src = _gather_rows(hx, f0)
    x = jnp.concatenate(
        [hx_src, pos1h,
         jnp.zeros((hx_src.shape[0], in_pad - in_sz), jnp.float32)], axis=1)
    a = _dense(x, w_all, b_all, None, False, be_e)
    h, c = _run_lstm(a, bgraph_flat, wh_all, wf_h, depth, nk, be_e)
    gn = _gather_rows(h, agraph_flat)
    nh = _sumcat_dense(hx, gn, enc["Wo"]["w"], enc["Wo"]["b"], nk, "relu", True, be_n)
    return nh, h


def kernel(tree_fnode, tree_fmess, tree_agraph, tree_bgraph, tree_cgraph,
           roots, graph_fnode, graph_fmess, graph_agraph, graph_bgraph, params):
    p = params
    depth = 3
    nei_g, nei_t, cw = 6, 6, 8

    genc = p["graph_encoder"]
    in_sz_g = 40 + 4 + 20
    w_all_g, b_all_g, wh_all_g, wf_h_g = _pack_lstm(genc["lstm"], in_sz_g, 128)
    src_atom = jnp.take(graph_fnode, graph_fmess[:, 0], axis=0)
    xg = jnp.concatenate(
        [jax.nn.one_hot(src_atom, 40, dtype=jnp.float32),
         jax.nn.one_hot(graph_fmess[:, 2], 4, dtype=jnp.float32),
         jax.nn.one_hot(graph_fmess[:, 3], 20, dtype=jnp.float32),
         jnp.zeros((graph_fmess.shape[0], 128 - in_sz_g), jnp.float32)], axis=1)
    a_g = _dense(xg, w_all_g, b_all_g, None, False, 1000)
    h_g, c_g = _run_lstm(a_g, graph_bgraph.reshape(-1), wh_all_g, wf_h_g,
                         depth, nei_g, 1000)
    gn_g = _gather_rows(h_g, graph_agraph.reshape(-1))
    fnode_g = jnp.pad(jax.nn.one_hot(graph_fnode, 40, dtype=jnp.float32),
                      ((0, 0), (0, 88)))
    wo_g = jnp.concatenate(
        [jnp.pad(genc["Wo"]["w"][:40], ((0, 88), (0, 0))), genc["Wo"]["w"][40:]],
        axis=0)
    hatom = _sumcat_dense(fnode_g, gn_g, wo_g, genc["Wo"]["b"], nei_g,
                          "relu", True, 1000)

    f0 = tree_fmess[:, 0]
    pos1h = jax.nn.one_hot(tree_fmess[:, 2], 20, dtype=jnp.float32)
    ag_flat = tree_agraph.reshape(-1)
    bg_flat = tree_bgraph.reshape(-1)

    hnode_b = _gather_rows(p["E_l"], tree_fnode[:, 2])
    hbond, _ = _tree_encoder(p["bond_encoder"], hnode_b, f0, pos1h,
                             ag_flat, bg_flat, depth, nei_t, 1000, 1000)

    finput1 = _gather_rows(p["E_i"], tree_fnode[:, 1])
    g_cg = _gather_rows(hatom, tree_cgraph.reshape(-1))
    hn = _sumcat_dense(finput1, g_cg, p["W_i"]["w"], p["W_i"]["b"], cw,
                       "relu", False, 1000)
    hinter, _ = _tree_encoder(p["frag_encoder"], hn, f0, pos1h,
                              ag_flat, bg_flat, depth, nei_t, 1000, 1000)

    finput = _gather_rows(p["E_c"], tree_fnode[:, 0])
    xc = jnp.concatenate([finput, hinter, hbond], axis=1)
    hnode_i = _dense(xc, p["W_c"]["w"], p["W_c"]["b"], "relu", False, 1000)
    hnode, hmess = _tree_encoder(p["inter_encoder"], hnode_i, f0, pos1h,
                                 ag_flat, bg_flat, depth, nei_t, 1000, 1000)

    fnode_r = _gather_rows(hnode_i, roots, ch=8)
    agr = jnp.take(tree_agraph, roots, axis=0).reshape(-1)
    g_r = _gather_rows(hmess, agr, ch=8)
    hroot = _sumcat_dense(fnode_r, g_r, p["W_root"]["w"], p["W_root"]["b"],
                          nei_t, "tanh", False, 128)

    return hroot, hnode, hinter, hbond, hatom


# speedup vs baseline: 1.8506x; 1.3693x over previous
import functools

import jax
import jax.numpy as jnp
from jax import lax
from jax.experimental import pallas as pl
from jax.experimental.pallas import tpu as pltpu
from jax.experimental.pallas import tpu_sc as plsc

_HID = 256
_NC = 2
_NS = 16
_NW = _NC * _NS


def _gather_rows(table, idx, ch=128):
    n = idx.shape[0]
    d = table.shape[1]
    dt = table.dtype
    unit = _NW * ch
    n_pad = ((n + unit - 1) // unit) * unit
    if n_pad != n:
        idx = jnp.pad(idx, (0, n_pad - n))
    chunks = n_pad // unit

    def body(tab_ref, idx_ref, out_ref, idx_v, rows_v, sem):
        wid = lax.axis_index("s") * _NC + lax.axis_index("c")

        def step(t, carry):
            base = (wid * chunks + t) * ch
            pltpu.sync_copy(idx_ref.at[pl.ds(base, ch)], idx_v)
            pltpu.async_copy(tab_ref.at[idx_v], rows_v, sem).wait()
            pltpu.sync_copy(rows_v, out_ref.at[pl.ds(base, ch)])
            return carry

        lax.fori_loop(0, chunks, step, 0)

    mesh = plsc.VectorSubcoreMesh(core_axis_name="c", subcore_axis_name="s")
    out = pl.kernel(
        body,
        mesh=mesh,
        out_type=jax.ShapeDtypeStruct((n_pad, d), dt),
        scratch_types=[
            pltpu.VMEM((ch,), jnp.int32),
            pltpu.VMEM((ch, d), dt),
            pltpu.SemaphoreType.DMA,
        ],
    )(table, idx)
    return out[:n]


def _apply_act(y, act):
    if act == "relu":
        return jnp.maximum(y, 0.0)
    if act == "tanh":
        return jnp.tanh(y)
    return y


def _mask0(y, i_blk, be):
    row = i_blk * be + lax.broadcasted_iota(jnp.int32, (be, 1), 0)
    return jnp.where(row == 0, 0.0, y)


def _dense(x, w, b, act, mask0, be):
    n, k = x.shape
    m = w.shape[1]
    b2 = b.reshape(1, m)

    def body(x_ref, w_ref, b_ref, o_ref):
        y = jnp.dot(x_ref[...], w_ref[...], preferred_element_type=jnp.float32)
        y = y + b_ref[...]
        y = _apply_act(y, act)
        if mask0:
            y = _mask0(y, pl.program_id(0), be)
        o_ref[...] = y

    return pl.pallas_call(
        body,
        grid=(n // be,),
        in_specs=[
            pl.BlockSpec((be, k), lambda i: (i, 0)),
            pl.BlockSpec((k, m), lambda i: (0, 0)),
            pl.BlockSpec((1, m), lambda i: (0, 0)),
        ],
        out_specs=pl.BlockSpec((be, m), lambda i: (i, 0)),
        out_shape=jax.ShapeDtypeStruct((n, m), jnp.float32),
    )(x, w, b2)


def _pack_hc(h, c):
    hu = lax.bitcast_convert_type(h, jnp.uint32)
    cu = lax.bitcast_convert_type(c, jnp.uint32)
    pk = ((hu + 0x8000) & jnp.uint32(0xFFFF0000)) | ((cu + 0x8000) >> 16)
    return lax.bitcast_convert_type(pk, jnp.int32)


def _unpack_hc(g):
    gu = lax.bitcast_convert_type(g, jnp.uint32)
    hv = lax.bitcast_convert_type(gu & jnp.uint32(0xFFFF0000), jnp.float32)
    cv = lax.bitcast_convert_type(gu << 16, jnp.float32)
    return hv, cv


def _lstm_init(a, be):
    e = a.shape[0]
    h = _HID

    def body(a_ref, h_ref, hc_ref):
        av = a_ref[...]
        gi = jax.nn.sigmoid(av[:, 0 * h:1 * h])
        go = jax.nn.sigmoid(av[:, 1 * h:2 * h])
        gg = jnp.tanh(av[:, 3 * h:4 * h])
        c_new = gi * gg
        h_new = go * jnp.tanh(c_new)
        h_new = _mask0(h_new, pl.program_id(0), be)
        c_new = _mask0(c_new, pl.program_id(0), be)
        h_ref[...] = h_new
        hc_ref[...] = _pack_hc(h_new, c_new)

    return pl.pallas_call(
        body,
        grid=(e // be,),
        in_specs=[pl.BlockSpec((be, 4 * h), lambda i: (i, 0))],
        out_specs=[
            pl.BlockSpec((be, h), lambda i: (i, 0)),
            pl.BlockSpec((be, h), lambda i: (i, 0)),
        ],
        out_shape=[
            jax.ShapeDtypeStruct((e, h), jnp.float32),
            jax.ShapeDtypeStruct((e, h), jnp.int32),
        ],
    )(a)


def _lstm_step(a, ghc, wh_all, wf_h, nk, be):
    e = a.shape[0]
    h = _HID

    def body(a_ref, g_ref, wh_ref, wf_ref, h_ref, hc_ref):
        av = a_ref[...]
        ghv, gcv = _unpack_hc(g_ref[...])
        gcv = gcv.reshape(be, nk, h)
        hs = ghv.reshape(be, nk, h).sum(axis=1)
        zh = jnp.dot(hs, wh_ref[...], preferred_element_type=jnp.float32)
        gi = jax.nn.sigmoid(av[:, 0 * h:1 * h] + zh[:, 0 * h:1 * h])
        go = jax.nn.sigmoid(av[:, 1 * h:2 * h] + zh[:, 1 * h:2 * h])
        gg = jnp.tanh(av[:, 3 * h:4 * h] + zh[:, 2 * h:3 * h])
        fpre = jnp.dot(ghv, wf_ref[...], preferred_element_type=jnp.float32)
        gf = jax.nn.sigmoid(fpre.reshape(be, nk, h) + av[:, 2 * h:3 * h][:, None, :])
        c_new = gi * gg + (gf * gcv).sum(axis=1)
        h_new = go * jnp.tanh(c_new)
        h_new = _mask0(h_new, pl.program_id(0), be)
        c_new = _mask0(c_new, pl.program_id(0), be)
        h_ref[...] = h_new
        hc_ref[...] = _pack_hc(h_new, c_new)

    return pl.pallas_call(
        body,
        grid=(e // be,),
        in_specs=[
            pl.BlockSpec((be, 4 * h), lambda i: (i, 0)),
            pl.BlockSpec((be * nk, h), lambda i: (i, 0)),
            pl.BlockSpec((h, 3 * h), lambda i: (0, 0)),
            pl.BlockSpec((h, h), lambda i: (0, 0)),
        ],
        out_specs=[
            pl.BlockSpec((be, h), lambda i: (i, 0)),
            pl.BlockSpec((be, h), lambda i: (i, 0)),
        ],
        out_shape=[
            jax.ShapeDtypeStruct((e, h), jnp.float32),
            jax.ShapeDtypeStruct((e, h), jnp.int32),
        ],
    )(a, ghc, wh_all, wf_h)


def _sumcat_dense(x1, g, w, b, nk, act, mask0, be):
    n, d1 = x1.shape
    h = _HID
    m = w.shape[1]
    b2 = b.reshape(1, m)

    def body(x_ref, g_ref, w_ref, b_ref, o_ref):
        nei = g_ref[...].reshape(be, nk, h).sum(axis=1)
        xcat = jnp.concatenate([x_ref[...], nei], axis=1)
        y = jnp.dot(xcat, w_ref[...], preferred_element_type=jnp.float32)
        y = y + b_ref[...]
        y = _apply_act(y, act)
        if mask0:
            y = _mask0(y, pl.program_id(0), be)
        o_ref[...] = y

    return pl.pallas_call(
        body,
        grid=(n // be,),
        in_specs=[
            pl.BlockSpec((be, d1), lambda i: (i, 0)),
            pl.BlockSpec((be * nk, h), lambda i: (i, 0)),
            pl.BlockSpec((d1 + h, m), lambda i: (0, 0)),
            pl.BlockSpec((1, m), lambda i: (0, 0)),
        ],
        out_specs=pl.BlockSpec((be, m), lambda i: (i, 0)),
        out_shape=jax.ShapeDtypeStruct((n, m), jnp.float32),
    )(x1, g, w, b2)


def _pack_lstm(lstm, in_sz, in_pad):
    wi, wo, wf, wg = lstm["Wi"], lstm["Wog"], lstm["Wf"], lstm["W"]
    w_all = jnp.concatenate(
        [wi["w"][:in_sz], wo["w"][:in_sz], wf["w"][:in_sz], wg["w"][:in_sz]], axis=1)
    if in_pad != in_sz:
        w_all = jnp.pad(w_all, ((0, in_pad - in_sz), (0, 0)))
    b_all = jnp.concatenate([wi["b"], wo["b"], wf["b"], wg["b"]])
    wh_all = jnp.concatenate(
        [wi["w"][in_sz:], wo["w"][in_sz:], wg["w"][in_sz:]], axis=1)
    wf_h = wf["w"][in_sz:]
    return w_all, b_all, wh_all, wf_h


def _run_lstm(a, bgraph_flat, wh_all, wf_h, depth, nk, be):
    h, hc = _lstm_init(a, be)
    for _ in range(depth - 1):
        ghc = _gather_rows(hc, bgraph_flat)
        h, hc = _lstm_step(a, ghc, wh_all, wf_h, nk, be)
    return h


def _tree_encoder(enc, hx, f0, pos1h, agraph_flat, bgraph_flat, depth, nk, be_e, be_n):
    in_sz = _HID + pos1h.shape[1]
    in_pad = 384
    w_all, b_all, wh_all, wf_h = _pack_lstm(enc["lstm"], in_sz, in_pad)
    hx_src = _gather_rows(hx, f0)
    x = jnp.concatenate(
        [hx_src, pos1h,
         jnp.zeros((hx_src.shape[0], in_pad - in_sz), jnp.float32)], axis=1)
    a = _dense(x, w_all, b_all, None, False, be_e)
    h = _run_lstm(a, bgraph_flat, wh_all, wf_h, depth, nk, be_e)
    gn = _gather_rows(h, agraph_flat)
    nh = _sumcat_dense(hx, gn, enc["Wo"]["w"], enc["Wo"]["b"], nk, "relu", True, be_n)
    return nh, h


def kernel(tree_fnode, tree_fmess, tree_agraph, tree_bgraph, tree_cgraph,
           roots, graph_fnode, graph_fmess, graph_agraph, graph_bgraph, params):
    p = params
    depth = 3
    nei_g, nei_t, cw = 6, 6, 8

    genc = p["graph_encoder"]
    in_sz_g = 40 + 4 + 20
    w_all_g, b_all_g, wh_all_g, wf_h_g = _pack_lstm(genc["lstm"], in_sz_g, 128)
    src_atom = jnp.take(graph_fnode, graph_fmess[:, 0], axis=0)
    xg = jnp.concatenate(
        [jax.nn.one_hot(src_atom, 40, dtype=jnp.float32),
         jax.nn.one_hot(graph_fmess[:, 2], 4, dtype=jnp.float32),
         jax.nn.one_hot(graph_fmess[:, 3], 20, dtype=jnp.float32),
         jnp.zeros((graph_fmess.shape[0], 128 - in_sz_g), jnp.float32)], axis=1)
    a_g = _dense(xg, w_all_g, b_all_g, None, False, 1000)
    h_g = _run_lstm(a_g, graph_bgraph.reshape(-1), wh_all_g, wf_h_g,
                    depth, nei_g, 1000)
    gn_g = _gather_rows(h_g, graph_agraph.reshape(-1))
    fnode_g = jnp.pad(jax.nn.one_hot(graph_fnode, 40, dtype=jnp.float32),
                      ((0, 0), (0, 88)))
    wo_g = jnp.concatenate(
        [jnp.pad(genc["Wo"]["w"][:40], ((0, 88), (0, 0))), genc["Wo"]["w"][40:]],
        axis=0)
    hatom = _sumcat_dense(fnode_g, gn_g, wo_g, genc["Wo"]["b"], nei_g,
                          "relu", True, 1000)

    f0 = tree_fmess[:, 0]
    pos1h = jax.nn.one_hot(tree_fmess[:, 2], 20, dtype=jnp.float32)
    ag_flat = tree_agraph.reshape(-1)
    bg_flat = tree_bgraph.reshape(-1)

    hnode_b = _gather_rows(p["E_l"], tree_fnode[:, 2])
    hbond, _ = _tree_encoder(p["bond_encoder"], hnode_b, f0, pos1h,
                             ag_flat, bg_flat, depth, nei_t, 1000, 1000)

    finput1 = _gather_rows(p["E_i"], tree_fnode[:, 1])
    g_cg = _gather_rows(hatom, tree_cgraph.reshape(-1))
    hn = _sumcat_dense(finput1, g_cg, p["W_i"]["w"], p["W_i"]["b"], cw,
                       "relu", False, 1000)
    hinter, _ = _tree_encoder(p["frag_encoder"], hn, f0, pos1h,
                              ag_flat, bg_flat, depth, nei_t, 1000, 1000)

    finput = _gather_rows(p["E_c"], tree_fnode[:, 0])
    xc = jnp.concatenate([finput, hinter, hbond], axis=1)
    hnode_i = _dense(xc, p["W_c"]["w"], p["W_c"]["b"], "relu", False, 1000)
    hnode, hmess = _tree_encoder(p["inter_encoder"], hnode_i, f0, pos1h,
                                 ag_flat, bg_flat, depth, nei_t, 1000, 1000)

    fnode_r = _gather_rows(hnode_i, roots, ch=8)
    agr = jnp.take(tree_agraph, roots, axis=0).reshape(-1)
    g_r = _gather_rows(hmess, agr, ch=8)
    hroot = _sumcat_dense(fnode_r, g_r, p["W_root"]["w"], p["W_root"]["b"],
                          nei_t, "tanh", False, 128)

    return hroot, hnode, hinter, hbond, hatom


# speedup vs baseline: 2.0614x; 1.1139x over previous
import functools

import jax
import jax.numpy as jnp
from jax import lax
from jax.experimental import pallas as pl
from jax.experimental.pallas import tpu as pltpu
from jax.experimental.pallas import tpu_sc as plsc

_HID = 256
_NC = 2
_NS = 16
_NW = _NC * _NS


def _gather_rows(table, idx, ch=128):
    n = idx.shape[0]
    d = table.shape[1]
    dt = table.dtype
    unit = _NW * ch
    n_pad = ((n + unit - 1) // unit) * unit
    if n_pad != n:
        idx = jnp.pad(idx, (0, n_pad - n))
    chunks = n_pad // unit

    pairs = chunks // 2
    tail = chunks % 2

    def body(tab_ref, idx_ref, out_ref, i0, i1, r0, r1, si0, si1, sg, sw0, sw1):
        wid = lax.axis_index("s") * _NC + lax.axis_index("c")
        base = wid * chunks

        def idx_cp(t, iv, sem):
            return pltpu.make_async_copy(idx_ref.at[pl.ds((base + t) * ch, ch)],
                                         iv, sem)

        def wb_cp(t, rv, sem):
            return pltpu.make_async_copy(rv, out_ref.at[pl.ds((base + t) * ch, ch)],
                                         sem)

        idx_cp(0, i0, si0).start()
        if chunks > 1:
            idx_cp(1, i1, si1).start()

        def half(t, iv, siv, rv, swv, p):
            t = jnp.int32(t)
            p = jnp.int32(p)
            idx_cp(t, iv, siv).wait()

            @pl.when(p > 0)
            def _():
                wb_cp(t, rv, swv).wait()

            pltpu.async_copy(tab_ref.at[iv], rv, sg).wait()
            wb_cp(t, rv, swv).start()

            @pl.when(t + 2 < chunks)
            def _():
                idx_cp(t + 2, iv, siv).start()

        def pair(p, carry):
            half(2 * p, i0, si0, r0, sw0, p)
            half(2 * p + 1, i1, si1, r1, sw1, p)
            return carry

        if pairs > 0:
            lax.fori_loop(0, pairs, pair, 0)
        if tail:
            half(chunks - 1, i0, si0, r0, sw0, pairs)
        last = chunks - 1
        if chunks >= 2:
            prev = last - 1
            wb_cp(prev, r0 if prev % 2 == 0 else r1,
                  sw0 if prev % 2 == 0 else sw1).wait()
        wb_cp(last, r0 if last % 2 == 0 else r1,
              sw0 if last % 2 == 0 else sw1).wait()

    mesh = plsc.VectorSubcoreMesh(core_axis_name="c", subcore_axis_name="s")
    out = pl.kernel(
        body,
        mesh=mesh,
        out_type=jax.ShapeDtypeStruct((n_pad, d), dt),
        scratch_types=[
            pltpu.VMEM((ch,), jnp.int32),
            pltpu.VMEM((ch,), jnp.int32),
            pltpu.VMEM((ch, d), dt),
            pltpu.VMEM((ch, d), dt),
            pltpu.SemaphoreType.DMA,
            pltpu.SemaphoreType.DMA,
            pltpu.SemaphoreType.DMA,
            pltpu.SemaphoreType.DMA,
            pltpu.SemaphoreType.DMA,
        ],
    )(table, idx)
    return out[:n]


def _apply_act(y, act):
    if act == "relu":
        return jnp.maximum(y, 0.0)
    if act == "tanh":
        return jnp.tanh(y)
    return y


def _mask0(y, i_blk, be):
    row = i_blk * be + lax.broadcasted_iota(jnp.int32, (be, 1), 0)
    return jnp.where(row == 0, 0.0, y)


def _dense(x, w, b, act, mask0, be):
    n, k = x.shape
    m = w.shape[1]
    b2 = b.reshape(1, m)

    def body(x_ref, w_ref, b_ref, o_ref):
        y = jnp.dot(x_ref[...], w_ref[...], preferred_element_type=jnp.float32)
        y = y + b_ref[...]
        y = _apply_act(y, act)
        if mask0:
            y = _mask0(y, pl.program_id(0), be)
        o_ref[...] = y

    return pl.pallas_call(
        body,
        grid=(n // be,),
        in_specs=[
            pl.BlockSpec((be, k), lambda i: (i, 0)),
            pl.BlockSpec((k, m), lambda i: (0, 0)),
            pl.BlockSpec((1, m), lambda i: (0, 0)),
        ],
        out_specs=pl.BlockSpec((be, m), lambda i: (i, 0)),
        out_shape=jax.ShapeDtypeStruct((n, m), jnp.float32),
    )(x, w, b2)


def _pack_hc(h, c):
    hu = lax.bitcast_convert_type(h, jnp.uint32)
    cu = lax.bitcast_convert_type(c, jnp.uint32)
    pk = ((hu + 0x8000) & jnp.uint32(0xFFFF0000)) | ((cu + 0x8000) >> 16)
    return lax.bitcast_convert_type(pk, jnp.int32)


def _unpack_hc(g):
    gu = lax.bitcast_convert_type(g, jnp.uint32)
    hv = lax.bitcast_convert_type(gu & jnp.uint32(0xFFFF0000), jnp.float32)
    cv = lax.bitcast_convert_type(gu << 16, jnp.float32)
    return hv, cv


def _lstm_init(a, be):
    e = a.shape[0]
    h = _HID

    def body(a_ref, h_ref, hc_ref):
        av = a_ref[...]
        gi = jax.nn.sigmoid(av[:, 0 * h:1 * h])
        go = jax.nn.sigmoid(av[:, 1 * h:2 * h])
        gg = jnp.tanh(av[:, 3 * h:4 * h])
        c_new = gi * gg
        h_new = go * jnp.tanh(c_new)
        h_new = _mask0(h_new, pl.program_id(0), be)
        c_new = _mask0(c_new, pl.program_id(0), be)
        h_ref[...] = h_new
        hc_ref[...] = _pack_hc(h_new, c_new)

    return pl.pallas_call(
        body,
        grid=(e // be,),
        in_specs=[pl.BlockSpec((be, 4 * h), lambda i: (i, 0))],
        out_specs=[
            pl.BlockSpec((be, h), lambda i: (i, 0)),
            pl.BlockSpec((be, h), lambda i: (i, 0)),
        ],
        out_shape=[
            jax.ShapeDtypeStruct((e, h), jnp.float32),
            jax.ShapeDtypeStruct((e, h), jnp.int32),
        ],
    )(a)


def _lstm_step(a, ghc, wh_all, wf_h, nk, be):
    e = a.shape[0]
    h = _HID

    def body(a_ref, g_ref, wh_ref, wf_ref, h_ref, hc_ref):
        av = a_ref[...]
        ghv, gcv = _unpack_hc(g_ref[...])
        gcv = gcv.reshape(be, nk, h)
        hs = ghv.reshape(be, nk, h).sum(axis=1)
        zh = jnp.dot(hs, wh_ref[...], preferred_element_type=jnp.float32)
        gi = jax.nn.sigmoid(av[:, 0 * h:1 * h] + zh[:, 0 * h:1 * h])
        go = jax.nn.sigmoid(av[:, 1 * h:2 * h] + zh[:, 1 * h:2 * h])
        gg = jnp.tanh(av[:, 3 * h:4 * h] + zh[:, 2 * h:3 * h])
        fpre = jnp.dot(ghv, wf_ref[...], preferred_element_type=jnp.float32)
        gf = jax.nn.sigmoid(fpre.reshape(be, nk, h) + av[:, 2 * h:3 * h][:, None, :])
        c_new = gi * gg + (gf * gcv).sum(axis=1)
        h_new = go * jnp.tanh(c_new)
        h_new = _mask0(h_new, pl.program_id(0), be)
        c_new = _mask0(c_new, pl.program_id(0), be)
        h_ref[...] = h_new
        hc_ref[...] = _pack_hc(h_new, c_new)

    return pl.pallas_call(
        body,
        grid=(e // be,),
        in_specs=[
            pl.BlockSpec((be, 4 * h), lambda i: (i, 0)),
            pl.BlockSpec((be * nk, h), lambda i: (i, 0)),
            pl.BlockSpec((h, 3 * h), lambda i: (0, 0)),
            pl.BlockSpec((h, h), lambda i: (0, 0)),
        ],
        out_specs=[
            pl.BlockSpec((be, h), lambda i: (i, 0)),
            pl.BlockSpec((be, h), lambda i: (i, 0)),
        ],
        out_shape=[
            jax.ShapeDtypeStruct((e, h), jnp.float32),
            jax.ShapeDtypeStruct((e, h), jnp.int32),
        ],
    )(a, ghc, wh_all, wf_h)


def _sumcat_dense(x1, g, w, b, nk, act, mask0, be):
    n, d1 = x1.shape
    h = _HID
    m = w.shape[1]
    b2 = b.reshape(1, m)

    def body(x_ref, g_ref, w_ref, b_ref, o_ref):
        nei = g_ref[...].reshape(be, nk, h).sum(axis=1)
        xcat = jnp.concatenate([x_ref[...], nei], axis=1)
        y = jnp.dot(xcat, w_ref[...], preferred_element_type=jnp.float32)
        y = y + b_ref[...]
        y = _apply_act(y, act)
        if mask0:
            y = _mask0(y, pl.program_id(0), be)
        o_ref[...] = y

    return pl.pallas_call(
        body,
        grid=(n // be,),
        in_specs=[
            pl.BlockSpec((be, d1), lambda i: (i, 0)),
            pl.BlockSpec((be * nk, h), lambda i: (i, 0)),
            pl.BlockSpec((d1 + h, m), lambda i: (0, 0)),
            pl.BlockSpec((1, m), lambda i: (0, 0)),
        ],
        out_specs=pl.BlockSpec((be, m), lambda i: (i, 0)),
        out_shape=jax.ShapeDtypeStruct((n, m), jnp.float32),
    )(x1, g, w, b2)


def _pack_lstm(lstm, in_sz, in_pad):
    wi, wo, wf, wg = lstm["Wi"], lstm["Wog"], lstm["Wf"], lstm["W"]
    w_all = jnp.concatenate(
        [wi["w"][:in_sz], wo["w"][:in_sz], wf["w"][:in_sz], wg["w"][:in_sz]], axis=1)
    if in_pad != in_sz:
        w_all = jnp.pad(w_all, ((0, in_pad - in_sz), (0, 0)))
    b_all = jnp.concatenate([wi["b"], wo["b"], wf["b"], wg["b"]])
    wh_all = jnp.concatenate(
        [wi["w"][in_sz:], wo["w"][in_sz:], wg["w"][in_sz:]], axis=1)
    wf_h = wf["w"][in_sz:]
    return w_all, b_all, wh_all, wf_h


def _run_lstm(a, bgraph_flat, wh_all, wf_h, depth, nk, be):
    h, hc = _lstm_init(a, be)
    for _ in range(depth - 1):
        ghc = _gather_rows(hc, bgraph_flat)
        h, hc = _lstm_step(a, ghc, wh_all, wf_h, nk, be)
    return h


def _tree_encoder(enc, hx, f0, pos1h, agraph_flat, bgraph_flat, depth, nk, be_e, be_n):
    in_sz = _HID + pos1h.shape[1]
    in_pad = 384
    w_all, b_all, wh_all, wf_h = _pack_lstm(enc["lstm"], in_sz, in_pad)
    hx_src = _gather_rows(hx, f0)
    x = jnp.concatenate(
        [hx_src, pos1h,
         jnp.zeros((hx_src.shape[0], in_pad - in_sz), jnp.float32)], axis=1)
    a = _dense(x, w_all, b_all, None, False, be_e)
    h = _run_lstm(a, bgraph_flat, wh_all, wf_h, depth, nk, be_e)
    gn = _gather_rows(h, agraph_flat)
    nh = _sumcat_dense(hx, gn, enc["Wo"]["w"], enc["Wo"]["b"], nk, "relu", True, be_n)
    return nh, h


def kernel(tree_fnode, tree_fmess, tree_agraph, tree_bgraph, tree_cgraph,
           roots, graph_fnode, graph_fmess, graph_agraph, graph_bgraph, params):
    p = params
    depth = 3
    nei_g, nei_t, cw = 6, 6, 8

    genc = p["graph_encoder"]
    in_sz_g = 40 + 4 + 20
    w_all_g, b_all_g, wh_all_g, wf_h_g = _pack_lstm(genc["lstm"], in_sz_g, 128)
    src_atom = jnp.take(graph_fnode, graph_fmess[:, 0], axis=0)
    xg = jnp.concatenate(
        [jax.nn.one_hot(src_atom, 40, dtype=jnp.float32),
         jax.nn.one_hot(graph_fmess[:, 2], 4, dtype=jnp.float32),
         jax.nn.one_hot(graph_fmess[:, 3], 20, dtype=jnp.float32),
         jnp.zeros((graph_fmess.shape[0], 128 - in_sz_g), jnp.float32)], axis=1)
    a_g = _dense(xg, w_all_g, b_all_g, None, False, 1000)
    h_g = _run_lstm(a_g, graph_bgraph.reshape(-1), wh_all_g, wf_h_g,
                    depth, nei_g, 1000)
    gn_g = _gather_rows(h_g, graph_agraph.reshape(-1))
    fnode_g = jnp.pad(jax.nn.one_hot(graph_fnode, 40, dtype=jnp.float32),
                      ((0, 0), (0, 88)))
    wo_g = jnp.concatenate(
        [jnp.pad(genc["Wo"]["w"][:40], ((0, 88), (0, 0))), genc["Wo"]["w"][40:]],
        axis=0)
    hatom = _sumcat_dense(fnode_g, gn_g, wo_g, genc["Wo"]["b"], nei_g,
                          "relu", True, 1000)

    f0 = tree_fmess[:, 0]
    pos1h = jax.nn.one_hot(tree_fmess[:, 2], 20, dtype=jnp.float32)
    ag_flat = tree_agraph.reshape(-1)
    bg_flat = tree_bgraph.reshape(-1)

    emb_tab = jnp.concatenate([p["E_l"], p["E_i"], p["E_c"]], axis=0)
    nl, ni = p["E_l"].shape[0], p["E_i"].shape[0]
    emb_idx = jnp.concatenate(
        [tree_fnode[:, 2], nl + tree_fnode[:, 1], nl + ni + tree_fnode[:, 0]])
    nt = tree_fnode.shape[0]
    emb = _gather_rows(emb_tab, emb_idx)
    hnode_b, finput1, finput = emb[:nt], emb[nt:2 * nt], emb[2 * nt:3 * nt]

    hbond, _ = _tree_encoder(p["bond_encoder"], hnode_b, f0, pos1h,
                             ag_flat, bg_flat, depth, nei_t, 1000, 1000)

    g_cg = _gather_rows(hatom, tree_cgraph.reshape(-1))
    hn = _sumcat_dense(finput1, g_cg, p["W_i"]["w"], p["W_i"]["b"], cw,
                       "relu", False, 1000)
    hinter, _ = _tree_encoder(p["frag_encoder"], hn, f0, pos1h,
                              ag_flat, bg_flat, depth, nei_t, 1000, 1000)

    xc = jnp.concatenate([finput, hinter, hbond], axis=1)
    hnode_i = _dense(xc, p["W_c"]["w"], p["W_c"]["b"], "relu", False, 1000)
    hnode, hmess = _tree_encoder(p["inter_encoder"], hnode_i, f0, pos1h,
                                 ag_flat, bg_flat, depth, nei_t, 1000, 1000)

    agr = jnp.take(tree_agraph, roots, axis=0).reshape(-1)
    root_tab = jnp.concatenate([hnode_i, hmess], axis=0)
    root_idx = jnp.concatenate([roots, hnode_i.shape[0] + agr])
    rg = _gather_rows(root_tab, root_idx, ch=8)
    nr = roots.shape[0]
    fnode_r, g_r = rg[:nr], rg[nr:nr + agr.shape[0]]
    hroot = _sumcat_dense(fnode_r, g_r, p["W_root"]["w"], p["W_root"]["b"],
                          nei_t, "tanh", False, 128)

    return hroot, hnode, hinter, hbond, hatom


# speedup vs baseline: 2.0705x; 1.0044x over previous
import functools

import jax
import jax.numpy as jnp
from jax import lax
from jax.experimental import pallas as pl
from jax.experimental.pallas import tpu as pltpu
from jax.experimental.pallas import tpu_sc as plsc

_HID = 256
_NC = 2
_NS = 16
_NW = _NC * _NS


def _gather_rows(table, idx, ch=128):
    n = idx.shape[0]
    d = table.shape[1]
    dt = table.dtype
    unit = _NW * ch
    n_pad = ((n + unit - 1) // unit) * unit
    if n_pad != n:
        idx = jnp.pad(idx, (0, n_pad - n))
    chunks = n_pad // unit

    pairs = chunks // 2
    tail = chunks % 2

    def body(tab_ref, idx_ref, out_ref, i0, i1, r0, r1, si0, si1, sg, sg1, sw0, sw1):
        wid = lax.axis_index("s") * _NC + lax.axis_index("c")
        base = wid * chunks

        def idx_cp(t, iv, sem):
            return pltpu.make_async_copy(idx_ref.at[pl.ds((base + t) * ch, ch)],
                                         iv, sem)

        def wb_cp(t, rv, sem):
            return pltpu.make_async_copy(rv, out_ref.at[pl.ds((base + t) * ch, ch)],
                                         sem)

        idx_cp(0, i0, si0).start()
        if chunks > 1:
            idx_cp(1, i1, si1).start()

        def g_cp(iv, rv, sem):
            return pltpu.make_async_copy(tab_ref.at[iv], rv, sem)

        def fire(t, iv, siv, rv, swv, sgv, p):
            idx_cp(t, iv, siv).wait()

            @pl.when(p > 0)
            def _():
                wb_cp(t, rv, swv).wait()

            g_cp(iv, rv, sgv).start()

        def drain(t, iv, siv, rv, swv, sgv):
            g_cp(iv, rv, sgv).wait()
            wb_cp(t, rv, swv).start()

            @pl.when(t + 2 < chunks)
            def _():
                idx_cp(t + 2, iv, siv).start()

        def pair(p, carry):
            t0 = jnp.int32(2 * p)
            t1 = t0 + 1
            fire(t0, i0, si0, r0, sw0, sg, p)
            fire(t1, i1, si1, r1, sw1, sg1, p)
            drain(t0, i0, si0, r0, sw0, sg)
            drain(t1, i1, si1, r1, sw1, sg1)
            return carry

        if pairs > 0:
            lax.fori_loop(0, pairs, pair, 0)
        if tail:
            t = jnp.int32(chunks - 1)
            fire(t, i0, si0, r0, sw0, sg, jnp.int32(pairs))
            drain(t, i0, si0, r0, sw0, sg)
        last = chunks - 1
        if chunks >= 2:
            prev = last - 1
            wb_cp(prev, r0 if prev % 2 == 0 else r1,
                  sw0 if prev % 2 == 0 else sw1).wait()
        wb_cp(last, r0 if last % 2 == 0 else r1,
              sw0 if last % 2 == 0 else sw1).wait()

    mesh = plsc.VectorSubcoreMesh(core_axis_name="c", subcore_axis_name="s")
    out = pl.kernel(
        body,
        mesh=mesh,
        out_type=jax.ShapeDtypeStruct((n_pad, d), dt),
        scratch_types=[
            pltpu.VMEM((ch,), jnp.int32),
            pltpu.VMEM((ch,), jnp.int32),
            pltpu.VMEM((ch, d), dt),
            pltpu.VMEM((ch, d), dt),
            pltpu.SemaphoreType.DMA,
            pltpu.SemaphoreType.DMA,
            pltpu.SemaphoreType.DMA,
            pltpu.SemaphoreType.DMA,
            pltpu.SemaphoreType.DMA,
            pltpu.SemaphoreType.DMA,
        ],
    )(table, idx)
    return out[:n]


def _apply_act(y, act):
    if act == "relu":
        return jnp.maximum(y, 0.0)
    if act == "tanh":
        return jnp.tanh(y)
    return y


def _mask0(y, i_blk, be):
    row = i_blk * be + lax.broadcasted_iota(jnp.int32, (be, 1), 0)
    return jnp.where(row == 0, 0.0, y)


def _dense(x, w, b, act, mask0, be):
    n, k = x.shape
    m = w.shape[1]
    b2 = b.reshape(1, m)

    def body(x_ref, w_ref, b_ref, o_ref):
        y = jnp.dot(x_ref[...], w_ref[...], preferred_element_type=jnp.float32)
        y = y + b_ref[...]
        y = _apply_act(y, act)
        if mask0:
            y = _mask0(y, pl.program_id(0), be)
        o_ref[...] = y

    return pl.pallas_call(
        body,
        grid=(n // be,),
        in_specs=[
            pl.BlockSpec((be, k), lambda i: (i, 0)),
            pl.BlockSpec((k, m), lambda i: (0, 0)),
            pl.BlockSpec((1, m), lambda i: (0, 0)),
        ],
        out_specs=pl.BlockSpec((be, m), lambda i: (i, 0)),
        out_shape=jax.ShapeDtypeStruct((n, m), jnp.float32),
    )(x, w, b2)


def _pack_hc(h, c):
    hu = lax.bitcast_convert_type(h, jnp.uint32)
    cu = lax.bitcast_convert_type(c, jnp.uint32)
    pk = ((hu + 0x8000) & jnp.uint32(0xFFFF0000)) | ((cu + 0x8000) >> 16)
    return lax.bitcast_convert_type(pk, jnp.int32)


def _unpack_hc(g):
    gu = lax.bitcast_convert_type(g, jnp.uint32)
    hv = lax.bitcast_convert_type(gu & jnp.uint32(0xFFFF0000), jnp.float32)
    cv = lax.bitcast_convert_type(gu << 16, jnp.float32)
    return hv, cv


def _lstm_init(a, be):
    e = a.shape[0]
    h = _HID

    def body(a_ref, h_ref, hc_ref):
        av = a_ref[...]
        gi = jax.nn.sigmoid(av[:, 0 * h:1 * h])
        go = jax.nn.sigmoid(av[:, 1 * h:2 * h])
        gg = jnp.tanh(av[:, 3 * h:4 * h])
        c_new = gi * gg
        h_new = go * jnp.tanh(c_new)
        h_new = _mask0(h_new, pl.program_id(0), be)
        c_new = _mask0(c_new, pl.program_id(0), be)
        h_ref[...] = h_new
        hc_ref[...] = _pack_hc(h_new, c_new)

    return pl.pallas_call(
        body,
        grid=(e // be,),
        in_specs=[pl.BlockSpec((be, 4 * h), lambda i: (i, 0))],
        out_specs=[
            pl.BlockSpec((be, h), lambda i: (i, 0)),
            pl.BlockSpec((be, h), lambda i: (i, 0)),
        ],
        out_shape=[
            jax.ShapeDtypeStruct((e, h), jnp.float32),
            jax.ShapeDtypeStruct((e, h), jnp.int32),
        ],
    )(a)


def _lstm_step(a, ghc, wh_all, wf_h, nk, be):
    e = a.shape[0]
    h = _HID

    def body(a_ref, g_ref, wh_ref, wf_ref, h_ref, hc_ref):
        av = a_ref[...]
        ghv, gcv = _unpack_hc(g_ref[...])
        gcv = gcv.reshape(be, nk, h)
        hs = ghv.reshape(be, nk, h).sum(axis=1)
        zh = jnp.dot(hs, wh_ref[...], preferred_element_type=jnp.float32)
        gi = jax.nn.sigmoid(av[:, 0 * h:1 * h] + zh[:, 0 * h:1 * h])
        go = jax.nn.sigmoid(av[:, 1 * h:2 * h] + zh[:, 1 * h:2 * h])
        gg = jnp.tanh(av[:, 3 * h:4 * h] + zh[:, 2 * h:3 * h])
        fpre = jnp.dot(ghv, wf_ref[...], preferred_element_type=jnp.float32)
        gf = jax.nn.sigmoid(fpre.reshape(be, nk, h) + av[:, 2 * h:3 * h][:, None, :])
        c_new = gi * gg + (gf * gcv).sum(axis=1)
        h_new = go * jnp.tanh(c_new)
        h_new = _mask0(h_new, pl.program_id(0), be)
        c_new = _mask0(c_new, pl.program_id(0), be)
        h_ref[...] = h_new
        hc_ref[...] = _pack_hc(h_new, c_new)

    return pl.pallas_call(
        body,
        grid=(e // be,),
        in_specs=[
            pl.BlockSpec((be, 4 * h), lambda i: (i, 0)),
            pl.BlockSpec((be * nk, h), lambda i: (i, 0)),
            pl.BlockSpec((h, 3 * h), lambda i: (0, 0)),
            pl.BlockSpec((h, h), lambda i: (0, 0)),
        ],
        out_specs=[
            pl.BlockSpec((be, h), lambda i: (i, 0)),
            pl.BlockSpec((be, h), lambda i: (i, 0)),
        ],
        out_shape=[
            jax.ShapeDtypeStruct((e, h), jnp.float32),
            jax.ShapeDtypeStruct((e, h), jnp.int32),
        ],
    )(a, ghc, wh_all, wf_h)


def _sumcat_dense(x1, g, w, b, nk, act, mask0, be):
    n, d1 = x1.shape
    h = _HID
    m = w.shape[1]
    b2 = b.reshape(1, m)

    def body(x_ref, g_ref, w_ref, b_ref, o_ref):
        nei = g_ref[...].reshape(be, nk, h).sum(axis=1)
        xcat = jnp.concatenate([x_ref[...], nei], axis=1)
        y = jnp.dot(xcat, w_ref[...], preferred_element_type=jnp.float32)
        y = y + b_ref[...]
        y = _apply_act(y, act)
        if mask0:
            y = _mask0(y, pl.program_id(0), be)
        o_ref[...] = y

    return pl.pallas_call(
        body,
        grid=(n // be,),
        in_specs=[
            pl.BlockSpec((be, d1), lambda i: (i, 0)),
            pl.BlockSpec((be * nk, h), lambda i: (i, 0)),
            pl.BlockSpec((d1 + h, m), lambda i: (0, 0)),
            pl.BlockSpec((1, m), lambda i: (0, 0)),
        ],
        out_specs=pl.BlockSpec((be, m), lambda i: (i, 0)),
        out_shape=jax.ShapeDtypeStruct((n, m), jnp.float32),
    )(x1, g, w, b2)


def _pack_lstm(lstm, in_sz, in_pad):
    wi, wo, wf, wg = lstm["Wi"], lstm["Wog"], lstm["Wf"], lstm["W"]
    w_all = jnp.concatenate(
        [wi["w"][:in_sz], wo["w"][:in_sz], wf["w"][:in_sz], wg["w"][:in_sz]], axis=1)
    if in_pad != in_sz:
        w_all = jnp.pad(w_all, ((0, in_pad - in_sz), (0, 0)))
    b_all = jnp.concatenate([wi["b"], wo["b"], wf["b"], wg["b"]])
    wh_all = jnp.concatenate(
        [wi["w"][in_sz:], wo["w"][in_sz:], wg["w"][in_sz:]], axis=1)
    wf_h = wf["w"][in_sz:]
    return w_all, b_all, wh_all, wf_h


def _run_lstm(a, bgraph_flat, wh_all, wf_h, depth, nk, be):
    h, hc = _lstm_init(a, be)
    for _ in range(depth - 1):
        ghc = _gather_rows(hc, bgraph_flat)
        h, hc = _lstm_step(a, ghc, wh_all, wf_h, nk, be)
    return h


def _tree_encoder(enc, hx, f0, pos1h, agraph_flat, bgraph_flat, depth, nk, be_e, be_n):
    in_sz = _HID + pos1h.shape[1]
    in_pad = 384
    w_all, b_all, wh_all, wf_h = _pack_lstm(enc["lstm"], in_sz, in_pad)
    hx_src = _gather_rows(hx, f0)
    x = jnp.concatenate(
        [hx_src, pos1h,
         jnp.zeros((hx_src.shape[0], in_pad - in_sz), jnp.float32)], axis=1)
    a = _dense(x, w_all, b_all, None, False, be_e)
    h = _run_lstm(a, bgraph_flat, wh_all, wf_h, depth, nk, be_e)
    gn = _gather_rows(h, agraph_flat)
    nh = _sumcat_dense(hx, gn, enc["Wo"]["w"], enc["Wo"]["b"], nk, "relu", True, be_n)
    return nh, h


def kernel(tree_fnode, tree_fmess, tree_agraph, tree_bgraph, tree_cgraph,
           roots, graph_fnode, graph_fmess, graph_agraph, graph_bgraph, params):
    p = params
    depth = 3
    nei_g, nei_t, cw = 6, 6, 8

    genc = p["graph_encoder"]
    in_sz_g = 40 + 4 + 20
    w_all_g, b_all_g, wh_all_g, wf_h_g = _pack_lstm(genc["lstm"], in_sz_g, 128)
    src_atom = jnp.take(graph_fnode, graph_fmess[:, 0], axis=0)
    xg = jnp.concatenate(
        [jax.nn.one_hot(src_atom, 40, dtype=jnp.float32),
         jax.nn.one_hot(graph_fmess[:, 2], 4, dtype=jnp.float32),
         jax.nn.one_hot(graph_fmess[:, 3], 20, dtype=jnp.float32),
         jnp.zeros((graph_fmess.shape[0], 128 - in_sz_g), jnp.float32)], axis=1)
    a_g = _dense(xg, w_all_g, b_all_g, None, False, 1000)
    h_g = _run_lstm(a_g, graph_bgraph.reshape(-1), wh_all_g, wf_h_g,
                    depth, nei_g, 1000)
    gn_g = _gather_rows(h_g, graph_agraph.reshape(-1))
    fnode_g = jnp.pad(jax.nn.one_hot(graph_fnode, 40, dtype=jnp.float32),
                      ((0, 0), (0, 88)))
    wo_g = jnp.concatenate(
        [jnp.pad(genc["Wo"]["w"][:40], ((0, 88), (0, 0))), genc["Wo"]["w"][40:]],
        axis=0)
    hatom = _sumcat_dense(fnode_g, gn_g, wo_g, genc["Wo"]["b"], nei_g,
                          "relu", True, 1000)

    f0 = tree_fmess[:, 0]
    pos1h = jax.nn.one_hot(tree_fmess[:, 2], 20, dtype=jnp.float32)
    ag_flat = tree_agraph.reshape(-1)
    bg_flat = tree_bgraph.reshape(-1)

    emb_tab = jnp.concatenate([p["E_l"], p["E_i"], p["E_c"]], axis=0)
    nl, ni = p["E_l"].shape[0], p["E_i"].shape[0]
    emb_idx = jnp.concatenate(
        [tree_fnode[:, 2], nl + tree_fnode[:, 1], nl + ni + tree_fnode[:, 0]])
    nt = tree_fnode.shape[0]
    emb = _gather_rows(emb_tab, emb_idx)
    hnode_b, finput1, finput = emb[:nt], emb[nt:2 * nt], emb[2 * nt:3 * nt]

    hbond, _ = _tree_encoder(p["bond_encoder"], hnode_b, f0, pos1h,
                             ag_flat, bg_flat, depth, nei_t, 1000, 1000)

    g_cg = _gather_rows(hatom, tree_cgraph.reshape(-1))
    hn = _sumcat_dense(finput1, g_cg, p["W_i"]["w"], p["W_i"]["b"], cw,
                       "relu", False, 1000)
    hinter, _ = _tree_encoder(p["frag_encoder"], hn, f0, pos1h,
                              ag_flat, bg_flat, depth, nei_t, 1000, 1000)

    xc = jnp.concatenate([finput, hinter, hbond], axis=1)
    hnode_i = _dense(xc, p["W_c"]["w"], p["W_c"]["b"], "relu", False, 1000)
    hnode, hmess = _tree_encoder(p["inter_encoder"], hnode_i, f0, pos1h,
                                 ag_flat, bg_flat, depth, nei_t, 1000, 1000)

    agr = jnp.take(tree_agraph, roots, axis=0).reshape(-1)
    root_tab = jnp.concatenate([hnode_i, hmess], axis=0)
    root_idx = jnp.concatenate([roots, hnode_i.shape[0] + agr])
    rg = _gather_rows(root_tab, root_idx, ch=8)
    nr = roots.shape[0]
    fnode_r, g_r = rg[:nr], rg[nr:nr + agr.shape[0]]
    hroot = _sumcat_dense(fnode_r, g_r, p["W_root"]["w"], p["W_root"]["b"],
                          nei_t, "tanh", False, 128)

    return hroot, hnode, hinter, hbond, hatom


# speedup vs baseline: 2.0968x; 1.0127x over previous
import functools

import jax
import jax.numpy as jnp
from jax import lax
from jax.experimental import pallas as pl
from jax.experimental.pallas import tpu as pltpu
from jax.experimental.pallas import tpu_sc as plsc

_HID = 256
_NC = 2
_NS = 16
_NW = _NC * _NS


def _gather_rows(table, idx, ch=128):
    n = idx.shape[0]
    d = table.shape[1]
    dt = table.dtype
    unit = _NW * ch
    n_pad = ((n + unit - 1) // unit) * unit
    if n_pad != n:
        idx = jnp.pad(idx, (0, n_pad - n))
    chunks = n_pad // unit

    pairs = chunks // 2
    tail = chunks % 2

    def body(tab_ref, idx_ref, out_ref, i0, i1, r0, r1, si0, si1, sg, sg1, sw0, sw1):
        wid = lax.axis_index("s") * _NC + lax.axis_index("c")
        base = wid * chunks

        def idx_cp(t, iv, sem):
            return pltpu.make_async_copy(idx_ref.at[pl.ds((base + t) * ch, ch)],
                                         iv, sem)

        def wb_cp(t, rv, sem):
            return pltpu.make_async_copy(rv, out_ref.at[pl.ds((base + t) * ch, ch)],
                                         sem)

        idx_cp(0, i0, si0).start()
        if chunks > 1:
            idx_cp(1, i1, si1).start()

        def g_cp(iv, rv, sem):
            return pltpu.make_async_copy(tab_ref.at[iv], rv, sem)

        def fire(t, iv, siv, rv, swv, sgv, p):
            idx_cp(t, iv, siv).wait()

            @pl.when(p > 0)
            def _():
                wb_cp(t, rv, swv).wait()

            g_cp(iv, rv, sgv).start()

        def drain(t, iv, siv, rv, swv, sgv):
            g_cp(iv, rv, sgv).wait()
            wb_cp(t, rv, swv).start()

            @pl.when(t + 2 < chunks)
            def _():
                idx_cp(t + 2, iv, siv).start()

        def pair(p, carry):
            t0 = jnp.int32(2 * p)
            t1 = t0 + 1
            fire(t0, i0, si0, r0, sw0, sg, p)
            fire(t1, i1, si1, r1, sw1, sg1, p)
            drain(t0, i0, si0, r0, sw0, sg)
            drain(t1, i1, si1, r1, sw1, sg1)
            return carry

        if pairs > 0:
            lax.fori_loop(0, pairs, pair, 0)
        if tail:
            t = jnp.int32(chunks - 1)
            fire(t, i0, si0, r0, sw0, sg, jnp.int32(pairs))
            drain(t, i0, si0, r0, sw0, sg)
        last = chunks - 1
        if chunks >= 2:
            prev = last - 1
            wb_cp(prev, r0 if prev % 2 == 0 else r1,
                  sw0 if prev % 2 == 0 else sw1).wait()
        wb_cp(last, r0 if last % 2 == 0 else r1,
              sw0 if last % 2 == 0 else sw1).wait()

    mesh = plsc.VectorSubcoreMesh(core_axis_name="c", subcore_axis_name="s")
    out = pl.kernel(
        body,
        mesh=mesh,
        out_type=jax.ShapeDtypeStruct((n_pad, d), dt),
        scratch_types=[
            pltpu.VMEM((ch,), jnp.int32),
            pltpu.VMEM((ch,), jnp.int32),
            pltpu.VMEM((ch, d), dt),
            pltpu.VMEM((ch, d), dt),
            pltpu.SemaphoreType.DMA,
            pltpu.SemaphoreType.DMA,
            pltpu.SemaphoreType.DMA,
            pltpu.SemaphoreType.DMA,
            pltpu.SemaphoreType.DMA,
            pltpu.SemaphoreType.DMA,
        ],
    )(table, idx)
    return out[:n]


def _apply_act(y, act):
    if act == "relu":
        return jnp.maximum(y, 0.0)
    if act == "tanh":
        return jnp.tanh(y)
    return y


def _mask0(y, i_blk, be):
    row = i_blk * be + lax.broadcasted_iota(jnp.int32, (be, 1), 0)
    return jnp.where(row == 0, 0.0, y)


def _dense(x, w, b, act, mask0, be):
    n, k = x.shape
    m = w.shape[1]
    b2 = b.reshape(1, m)

    def body(x_ref, w_ref, b_ref, o_ref):
        y = jnp.dot(x_ref[...], w_ref[...], preferred_element_type=jnp.float32)
        y = y + b_ref[...]
        y = _apply_act(y, act)
        if mask0:
            y = _mask0(y, pl.program_id(0), be)
        o_ref[...] = y

    return pl.pallas_call(
        body,
        grid=(n // be,),
        in_specs=[
            pl.BlockSpec((be, k), lambda i: (i, 0)),
            pl.BlockSpec((k, m), lambda i: (0, 0)),
            pl.BlockSpec((1, m), lambda i: (0, 0)),
        ],
        out_specs=pl.BlockSpec((be, m), lambda i: (i, 0)),
        out_shape=jax.ShapeDtypeStruct((n, m), jnp.float32),
    )(x, w, b2)


def _pack_hc(h, c):
    hu = lax.bitcast_convert_type(h, jnp.uint32)
    cu = lax.bitcast_convert_type(c, jnp.uint32)
    pk = ((hu + 0x8000) & jnp.uint32(0xFFFF0000)) | ((cu + 0x8000) >> 16)
    return lax.bitcast_convert_type(pk, jnp.int32)


def _unpack_hc(g):
    gu = lax.bitcast_convert_type(g, jnp.uint32)
    hv = lax.bitcast_convert_type(gu & jnp.uint32(0xFFFF0000), jnp.float32)
    cv = lax.bitcast_convert_type(gu << 16, jnp.float32)
    return hv, cv


def _pack_cols(y):
    return _pack_hc(y[:, :_HID // 2], y[:, _HID // 2:])


def _unpack_cols(g):
    y1, y2 = _unpack_hc(g)
    return jnp.concatenate([y1, y2], axis=1)


def _lstm_init(a, be):
    e = a.shape[0]
    h = _HID

    def body(a_ref, h_ref, hc_ref):
        av = a_ref[...]
        gi = jax.nn.sigmoid(av[:, 0 * h:1 * h])
        go = jax.nn.sigmoid(av[:, 1 * h:2 * h])
        gg = jnp.tanh(av[:, 3 * h:4 * h])
        c_new = gi * gg
        h_new = go * jnp.tanh(c_new)
        h_new = _mask0(h_new, pl.program_id(0), be)
        c_new = _mask0(c_new, pl.program_id(0), be)
        h_ref[...] = h_new
        hc_ref[...] = _pack_hc(h_new, c_new)

    return pl.pallas_call(
        body,
        grid=(e // be,),
        in_specs=[pl.BlockSpec((be, 4 * h), lambda i: (i, 0))],
        out_specs=[
            pl.BlockSpec((be, h), lambda i: (i, 0)),
            pl.BlockSpec((be, h), lambda i: (i, 0)),
        ],
        out_shape=[
            jax.ShapeDtypeStruct((e, h), jnp.float32),
            jax.ShapeDtypeStruct((e, h), jnp.int32),
        ],
    )(a)


def _lstm_step(a, ghc, wh_all, wf_h, nk, be, last):
    e = a.shape[0]
    h = _HID

    def body(a_ref, g_ref, wh_ref, wf_ref, h_ref, p_ref):
        av = a_ref[...]
        ghv, gcv = _unpack_hc(g_ref[...])
        gcv = gcv.reshape(be, nk, h)
        hs = ghv.reshape(be, nk, h).sum(axis=1)
        zh = jnp.dot(hs, wh_ref[...], preferred_element_type=jnp.float32)
        gi = jax.nn.sigmoid(av[:, 0 * h:1 * h] + zh[:, 0 * h:1 * h])
        go = jax.nn.sigmoid(av[:, 1 * h:2 * h] + zh[:, 1 * h:2 * h])
        gg = jnp.tanh(av[:, 3 * h:4 * h] + zh[:, 2 * h:3 * h])
        fpre = jnp.dot(ghv, wf_ref[...], preferred_element_type=jnp.float32)
        gf = jax.nn.sigmoid(fpre.reshape(be, nk, h) + av[:, 2 * h:3 * h][:, None, :])
        c_new = gi * gg + (gf * gcv).sum(axis=1)
        h_new = go * jnp.tanh(c_new)
        h_new = _mask0(h_new, pl.program_id(0), be)
        c_new = _mask0(c_new, pl.program_id(0), be)
        h_ref[...] = h_new
        if last:
            p_ref[...] = _pack_cols(h_new)
        else:
            p_ref[...] = _pack_hc(h_new, c_new)

    pw = h // 2 if last else h
    return pl.pallas_call(
        body,
        grid=(e // be,),
        in_specs=[
            pl.BlockSpec((be, 4 * h), lambda i: (i, 0)),
            pl.BlockSpec((be * nk, h), lambda i: (i, 0)),
            pl.BlockSpec((h, 3 * h), lambda i: (0, 0)),
            pl.BlockSpec((h, h), lambda i: (0, 0)),
        ],
        out_specs=[
            pl.BlockSpec((be, h), lambda i: (i, 0)),
            pl.BlockSpec((be, pw), lambda i: (i, 0)),
        ],
        out_shape=[
            jax.ShapeDtypeStruct((e, h), jnp.float32),
            jax.ShapeDtypeStruct((e, pw), jnp.int32),
        ],
    )(a, ghc, wh_all, wf_h)


def _sumcat_dense(x1, g, w, b, nk, act, mask0, be, packed_g=False, emit_pk=False):
    n, d1 = x1.shape
    h = _HID
    m = w.shape[1]
    b2 = b.reshape(1, m)
    gw = h // 2 if packed_g else h

    def body(x_ref, g_ref, w_ref, b_ref, *o_refs):
        gv = g_ref[...]
        if packed_g:
            gv = _unpack_cols(gv)
        nei = gv.reshape(be, nk, h).sum(axis=1)
        xcat = jnp.concatenate([x_ref[...], nei], axis=1)
        y = jnp.dot(xcat, w_ref[...], preferred_element_type=jnp.float32)
        y = y + b_ref[...]
        y = _apply_act(y, act)
        if mask0:
            y = _mask0(y, pl.program_id(0), be)
        o_refs[0][...] = y
        if emit_pk:
            o_refs[1][...] = _pack_cols(y)

    out_specs = [pl.BlockSpec((be, m), lambda i: (i, 0))]
    out_shape = [jax.ShapeDtypeStruct((n, m), jnp.float32)]
    if emit_pk:
        out_specs.append(pl.BlockSpec((be, m // 2), lambda i: (i, 0)))
        out_shape.append(jax.ShapeDtypeStruct((n, m // 2), jnp.int32))
    res = pl.pallas_call(
        body,
        grid=(n // be,),
        in_specs=[
            pl.BlockSpec((be, d1), lambda i: (i, 0)),
            pl.BlockSpec((be * nk, gw), lambda i: (i, 0)),
            pl.BlockSpec((d1 + h, m), lambda i: (0, 0)),
            pl.BlockSpec((1, m), lambda i: (0, 0)),
        ],
        out_specs=out_specs,
        out_shape=out_shape,
    )(x1, g, w, b2)
    return res if emit_pk else res[0]


def _pack_lstm(lstm, in_sz, in_pad):
    wi, wo, wf, wg = lstm["Wi"], lstm["Wog"], lstm["Wf"], lstm["W"]
    w_all = jnp.concatenate(
        [wi["w"][:in_sz], wo["w"][:in_sz], wf["w"][:in_sz], wg["w"][:in_sz]], axis=1)
    if in_pad != in_sz:
        w_all = jnp.pad(w_all, ((0, in_pad - in_sz), (0, 0)))
    b_all = jnp.concatenate([wi["b"], wo["b"], wf["b"], wg["b"]])
    wh_all = jnp.concatenate(
        [wi["w"][in_sz:], wo["w"][in_sz:], wg["w"][in_sz:]], axis=1)
    wf_h = wf["w"][in_sz:]
    return w_all, b_all, wh_all, wf_h


def _run_lstm(a, bgraph_flat, wh_all, wf_h, depth, nk, be):
    h, hc = _lstm_init(a, be)
    for it in range(depth - 1):
        ghc = _gather_rows(hc, bgraph_flat)
        h, hc = _lstm_step(a, ghc, wh_all, wf_h, nk, be, it == depth - 2)
    return h, hc


def _tree_encoder(enc, hx, f0, pos1h, agraph_flat, bgraph_flat, depth, nk, be_e, be_n):
    in_sz = _HID + pos1h.shape[1]
    in_pad = 384
    w_all, b_all, wh_all, wf_h = _pack_lstm(enc["lstm"], in_sz, in_pad)
    hx_src = _gather_rows(hx, f0)
    x = jnp.concatenate(
        [hx_src, pos1h,
         jnp.zeros((hx_src.shape[0], in_pad - in_sz), jnp.float32)], axis=1)
    a = _dense(x, w_all, b_all, None, False, be_e)
    h, hpk = _run_lstm(a, bgraph_flat, wh_all, wf_h, depth, nk, be_e)
    gn = _gather_rows(hpk, agraph_flat)
    nh = _sumcat_dense(hx, gn, enc["Wo"]["w"], enc["Wo"]["b"], nk, "relu", True,
                       be_n, packed_g=True)
    return nh, h


def kernel(tree_fnode, tree_fmess, tree_agraph, tree_bgraph, tree_cgraph,
           roots, graph_fnode, graph_fmess, graph_agraph, graph_bgraph, params):
    p = params
    depth = 3
    nei_g, nei_t, cw = 6, 6, 8

    genc = p["graph_encoder"]
    in_sz_g = 40 + 4 + 20
    w_all_g, b_all_g, wh_all_g, wf_h_g = _pack_lstm(genc["lstm"], in_sz_g, 128)
    src_atom = jnp.take(graph_fnode, graph_fmess[:, 0], axis=0)
    xg = jnp.concatenate(
        [jax.nn.one_hot(src_atom, 40, dtype=jnp.float32),
         jax.nn.one_hot(graph_fmess[:, 2], 4, dtype=jnp.float32),
         jax.nn.one_hot(graph_fmess[:, 3], 20, dtype=jnp.float32),
         jnp.zeros((graph_fmess.shape[0], 128 - in_sz_g), jnp.float32)], axis=1)
    a_g = _dense(xg, w_all_g, b_all_g, None, False, 1000)
    _, hpk_g = _run_lstm(a_g, graph_bgraph.reshape(-1), wh_all_g, wf_h_g,
                         depth, nei_g, 1000)
    gn_g = _gather_rows(hpk_g, graph_agraph.reshape(-1))
    fnode_g = jnp.pad(jax.nn.one_hot(graph_fnode, 40, dtype=jnp.float32),
                      ((0, 0), (0, 88)))
    wo_g = jnp.concatenate(
        [jnp.pad(genc["Wo"]["w"][:40], ((0, 88), (0, 0))), genc["Wo"]["w"][40:]],
        axis=0)
    hatom, hatom_pk = _sumcat_dense(fnode_g, gn_g, wo_g, genc["Wo"]["b"], nei_g,
                                    "relu", True, 1000, packed_g=True,
                                    emit_pk=True)

    f0 = tree_fmess[:, 0]
    pos1h = jax.nn.one_hot(tree_fmess[:, 2], 20, dtype=jnp.float32)
    ag_flat = tree_agraph.reshape(-1)
    bg_flat = tree_bgraph.reshape(-1)

    emb_tab = jnp.concatenate([p["E_l"], p["E_i"], p["E_c"]], axis=0)
    nl, ni = p["E_l"].shape[0], p["E_i"].shape[0]
    emb_idx = jnp.concatenate(
        [tree_fnode[:, 2], nl + tree_fnode[:, 1], nl + ni + tree_fnode[:, 0]])
    nt = tree_fnode.shape[0]
    emb = _gather_rows(emb_tab, emb_idx)
    hnode_b, finput1, finput = emb[:nt], emb[nt:2 * nt], emb[2 * nt:3 * nt]

    hbond, _ = _tree_encoder(p["bond_encoder"], hnode_b, f0, pos1h,
                             ag_flat, bg_flat, depth, nei_t, 1000, 1000)

    g_cg = _gather_rows(hatom_pk, tree_cgraph.reshape(-1))
    hn = _sumcat_dense(finput1, g_cg, p["W_i"]["w"], p["W_i"]["b"], cw,
                       "relu", False, 1000, packed_g=True)
    hinter, _ = _tree_encoder(p["frag_encoder"], hn, f0, pos1h,
                              ag_flat, bg_flat, depth, nei_t, 1000, 1000)

    xc = jnp.concatenate([finput, hinter, hbond], axis=1)
    hnode_i = _dense(xc, p["W_c"]["w"], p["W_c"]["b"], "relu", False, 1000)
    hnode, hmess = _tree_encoder(p["inter_encoder"], hnode_i, f0, pos1h,
                                 ag_flat, bg_flat, depth, nei_t, 1000, 1000)

    agr = jnp.take(tree_agraph, roots, axis=0).reshape(-1)
    root_tab = jnp.concatenate([hnode_i, hmess], axis=0)
    root_idx = jnp.concatenate([roots, hnode_i.shape[0] + agr])
    rg = _gather_rows(root_tab, root_idx, ch=8)
    nr = roots.shape[0]
    fnode_r, g_r = rg[:nr], rg[nr:nr + agr.shape[0]]
    hroot = _sumcat_dense(fnode_r, g_r, p["W_root"]["w"], p["W_root"]["b"],
                          nei_t, "tanh", False, 128)

    return hroot, hnode, hinter, hbond, hatom


# speedup vs baseline: 2.1509x; 1.0258x over previous
import functools

import jax
import jax.numpy as jnp
from jax import lax
from jax.experimental import pallas as pl
from jax.experimental.pallas import tpu as pltpu
from jax.experimental.pallas import tpu_sc as plsc

_HID = 256
_NC = 2
_NS = 16
_NW = _NC * _NS


def _gather_rows(table, idx, ch=128):
    n = idx.shape[0]
    d = table.shape[1]
    dt = table.dtype
    unit = _NW * ch
    n_pad = ((n + unit - 1) // unit) * unit
    if n_pad != n:
        idx = jnp.pad(idx, (0, n_pad - n))
    chunks = n_pad // unit

    pairs = chunks // 2
    tail = chunks % 2

    def body(tab_ref, idx_ref, out_ref, i0, i1, r0, r1, si0, si1, sg, sg1, sw0, sw1):
        wid = lax.axis_index("s") * _NC + lax.axis_index("c")
        base = wid * chunks

        def idx_cp(t, iv, sem):
            return pltpu.make_async_copy(idx_ref.at[pl.ds((base + t) * ch, ch)],
                                         iv, sem)

        def wb_cp(t, rv, sem):
            return pltpu.make_async_copy(rv, out_ref.at[pl.ds((base + t) * ch, ch)],
                                         sem)

        idx_cp(0, i0, si0).start()
        if chunks > 1:
            idx_cp(1, i1, si1).start()

        def g_cp(iv, rv, sem):
            return pltpu.make_async_copy(tab_ref.at[iv], rv, sem)

        def fire(t, iv, siv, rv, swv, sgv, p):
            idx_cp(t, iv, siv).wait()

            @pl.when(p > 0)
            def _():
                wb_cp(t, rv, swv).wait()

            g_cp(iv, rv, sgv).start()

        def drain(t, iv, siv, rv, swv, sgv):
            g_cp(iv, rv, sgv).wait()
            wb_cp(t, rv, swv).start()

            @pl.when(t + 2 < chunks)
            def _():
                idx_cp(t + 2, iv, siv).start()

        def pair(p, carry):
            t0 = jnp.int32(2 * p)
            t1 = t0 + 1
            fire(t0, i0, si0, r0, sw0, sg, p)
            fire(t1, i1, si1, r1, sw1, sg1, p)
            drain(t0, i0, si0, r0, sw0, sg)
            drain(t1, i1, si1, r1, sw1, sg1)
            return carry

        if pairs > 0:
            lax.fori_loop(0, pairs, pair, 0)
        if tail:
            t = jnp.int32(chunks - 1)
            fire(t, i0, si0, r0, sw0, sg, jnp.int32(pairs))
            drain(t, i0, si0, r0, sw0, sg)
        last = chunks - 1
        if chunks >= 2:
            prev = last - 1
            wb_cp(prev, r0 if prev % 2 == 0 else r1,
                  sw0 if prev % 2 == 0 else sw1).wait()
        wb_cp(last, r0 if last % 2 == 0 else r1,
              sw0 if last % 2 == 0 else sw1).wait()

    mesh = plsc.VectorSubcoreMesh(core_axis_name="c", subcore_axis_name="s")
    out = pl.kernel(
        body,
        mesh=mesh,
        out_type=jax.ShapeDtypeStruct((n_pad, d), dt),
        scratch_types=[
            pltpu.VMEM((ch,), jnp.int32),
            pltpu.VMEM((ch,), jnp.int32),
            pltpu.VMEM((ch, d), dt),
            pltpu.VMEM((ch, d), dt),
            pltpu.SemaphoreType.DMA,
            pltpu.SemaphoreType.DMA,
            pltpu.SemaphoreType.DMA,
            pltpu.SemaphoreType.DMA,
            pltpu.SemaphoreType.DMA,
            pltpu.SemaphoreType.DMA,
        ],
    )(table, idx)
    return out[:n]


def _apply_act(y, act):
    if act == "relu":
        return jnp.maximum(y, 0.0)
    if act == "tanh":
        return jnp.tanh(y)
    return y


def _mask0(y, i_blk, be, bpe):
    row = (i_blk % bpe) * be + lax.broadcasted_iota(jnp.int32, (be, 1), 0)
    return jnp.where(row == 0, 0.0, y)


def _dense(x, w, b, act, mask0, be, bpe=None):
    n, k = x.shape
    m = w.shape[1]
    if bpe is None:
        bpe = n // be
        w = w.reshape(1, k, m)
        b2 = b.reshape(1, 1, m)
    else:
        w = w.reshape(-1, k, m)
        b2 = b.reshape(-1, 1, m)

    def body(x_ref, w_ref, b_ref, o_ref):
        y = jnp.dot(x_ref[...], w_ref[0], preferred_element_type=jnp.float32)
        y = y + b_ref[0]
        y = _apply_act(y, act)
        if mask0:
            y = _mask0(y, pl.program_id(0), be, bpe)
        o_ref[...] = y

    return pl.pallas_call(
        body,
        grid=(n // be,),
        in_specs=[
            pl.BlockSpec((be, k), lambda i: (i, 0)),
            pl.BlockSpec((1, k, m), lambda i: (i // bpe, 0, 0)),
            pl.BlockSpec((1, 1, m), lambda i: (i // bpe, 0, 0)),
        ],
        out_specs=pl.BlockSpec((be, m), lambda i: (i, 0)),
        out_shape=jax.ShapeDtypeStruct((n, m), jnp.float32),
    )(x, w, b2)


def _pack_hc(h, c):
    hu = lax.bitcast_convert_type(h, jnp.uint32)
    cu = lax.bitcast_convert_type(c, jnp.uint32)
    pk = ((hu + 0x8000) & jnp.uint32(0xFFFF0000)) | ((cu + 0x8000) >> 16)
    return lax.bitcast_convert_type(pk, jnp.int32)


def _unpack_hc(g):
    gu = lax.bitcast_convert_type(g, jnp.uint32)
    hv = lax.bitcast_convert_type(gu & jnp.uint32(0xFFFF0000), jnp.float32)
    cv = lax.bitcast_convert_type(gu << 16, jnp.float32)
    return hv, cv


def _pack_cols(y):
    return _pack_hc(y[:, :_HID // 2], y[:, _HID // 2:])


def _unpack_cols(g):
    y1, y2 = _unpack_hc(g)
    return jnp.concatenate([y1, y2], axis=1)


def _lstm_init(a, be, bpe=None):
    e = a.shape[0]
    h = _HID
    if bpe is None:
        bpe = e // be

    def body(a_ref, h_ref, hc_ref):
        av = a_ref[...]
        gi = jax.nn.sigmoid(av[:, 0 * h:1 * h])
        go = jax.nn.sigmoid(av[:, 1 * h:2 * h])
        gg = jnp.tanh(av[:, 3 * h:4 * h])
        c_new = gi * gg
        h_new = go * jnp.tanh(c_new)
        h_new = _mask0(h_new, pl.program_id(0), be, bpe)
        c_new = _mask0(c_new, pl.program_id(0), be, bpe)
        h_ref[...] = h_new
        hc_ref[...] = _pack_hc(h_new, c_new)

    return pl.pallas_call(
        body,
        grid=(e // be,),
        in_specs=[pl.BlockSpec((be, 4 * h), lambda i: (i, 0))],
        out_specs=[
            pl.BlockSpec((be, h), lambda i: (i, 0)),
            pl.BlockSpec((be, h), lambda i: (i, 0)),
        ],
        out_shape=[
            jax.ShapeDtypeStruct((e, h), jnp.float32),
            jax.ShapeDtypeStruct((e, h), jnp.int32),
        ],
    )(a)


def _lstm_step(a, ghc, wh_all, wf_h, nk, be, last, bpe=None):
    e = a.shape[0]
    h = _HID
    if bpe is None:
        bpe = e // be
    wh_all = wh_all.reshape(-1, h, 3 * h)
    wf_h = wf_h.reshape(-1, h, h)

    def body(a_ref, g_ref, wh_ref, wf_ref, h_ref, p_ref):
        av = a_ref[...]
        ghv, gcv = _unpack_hc(g_ref[...])
        gcv = gcv.reshape(be, nk, h)
        hs = ghv.reshape(be, nk, h).sum(axis=1)
        zh = jnp.dot(hs, wh_ref[0], preferred_element_type=jnp.float32)
        gi = jax.nn.sigmoid(av[:, 0 * h:1 * h] + zh[:, 0 * h:1 * h])
        go = jax.nn.sigmoid(av[:, 1 * h:2 * h] + zh[:, 1 * h:2 * h])
        gg = jnp.tanh(av[:, 3 * h:4 * h] + zh[:, 2 * h:3 * h])
        fpre = jnp.dot(ghv, wf_ref[0], preferred_element_type=jnp.float32)
        gf = jax.nn.sigmoid(fpre.reshape(be, nk, h) + av[:, 2 * h:3 * h][:, None, :])
        c_new = gi * gg + (gf * gcv).sum(axis=1)
        h_new = go * jnp.tanh(c_new)
        h_new = _mask0(h_new, pl.program_id(0), be, bpe)
        c_new = _mask0(c_new, pl.program_id(0), be, bpe)
        h_ref[...] = h_new
        if last:
            p_ref[...] = _pack_cols(h_new)
        else:
            p_ref[...] = _pack_hc(h_new, c_new)

    pw = h // 2 if last else h
    return pl.pallas_call(
        body,
        grid=(e // be,),
        in_specs=[
            pl.BlockSpec((be, 4 * h), lambda i: (i, 0)),
            pl.BlockSpec((be * nk, h), lambda i: (i, 0)),
            pl.BlockSpec((1, h, 3 * h), lambda i: (i // bpe, 0, 0)),
            pl.BlockSpec((1, h, h), lambda i: (i // bpe, 0, 0)),
        ],
        out_specs=[
            pl.BlockSpec((be, h), lambda i: (i, 0)),
            pl.BlockSpec((be, pw), lambda i: (i, 0)),
        ],
        out_shape=[
            jax.ShapeDtypeStruct((e, h), jnp.float32),
            jax.ShapeDtypeStruct((e, pw), jnp.int32),
        ],
    )(a, ghc, wh_all, wf_h)


def _sumcat_dense(x1, g, w, b, nk, act, mask0, be, packed_g=False, emit_pk=False,
                  bpe=None):
    n, d1 = x1.shape
    h = _HID
    m = w.shape[1]
    if bpe is None:
        bpe = n // be
    w = w.reshape(-1, d1 + h, m)
    b2 = b.reshape(-1, 1, m)
    gw = h // 2 if packed_g else h

    def body(x_ref, g_ref, w_ref, b_ref, *o_refs):
        gv = g_ref[...]
        if packed_g:
            gv = _unpack_cols(gv)
        nei = gv.reshape(be, nk, h).sum(axis=1)
        xcat = jnp.concatenate([x_ref[...], nei], axis=1)
        y = jnp.dot(xcat, w_ref[0], preferred_element_type=jnp.float32)
        y = y + b_ref[0]
        y = _apply_act(y, act)
        if mask0:
            y = _mask0(y, pl.program_id(0), be, bpe)
        o_refs[0][...] = y
        if emit_pk:
            o_refs[1][...] = _pack_cols(y)

    out_specs = [pl.BlockSpec((be, m), lambda i: (i, 0))]
    out_shape = [jax.ShapeDtypeStruct((n, m), jnp.float32)]
    if emit_pk:
        out_specs.append(pl.BlockSpec((be, m // 2), lambda i: (i, 0)))
        out_shape.append(jax.ShapeDtypeStruct((n, m // 2), jnp.int32))
    res = pl.pallas_call(
        body,
        grid=(n // be,),
        in_specs=[
            pl.BlockSpec((be, d1), lambda i: (i, 0)),
            pl.BlockSpec((be * nk, gw), lambda i: (i, 0)),
            pl.BlockSpec((1, d1 + h, m), lambda i: (i // bpe, 0, 0)),
            pl.BlockSpec((1, 1, m), lambda i: (i // bpe, 0, 0)),
        ],
        out_specs=out_specs,
        out_shape=out_shape,
    )(x1, g, w, b2)
    return res if emit_pk else res[0]


def _pack_lstm(lstm, in_sz, in_pad):
    wi, wo, wf, wg = lstm["Wi"], lstm["Wog"], lstm["Wf"], lstm["W"]
    w_all = jnp.concatenate(
        [wi["w"][:in_sz], wo["w"][:in_sz], wf["w"][:in_sz], wg["w"][:in_sz]], axis=1)
    if in_pad != in_sz:
        w_all = jnp.pad(w_all, ((0, in_pad - in_sz), (0, 0)))
    b_all = jnp.concatenate([wi["b"], wo["b"], wf["b"], wg["b"]])
    wh_all = jnp.concatenate(
        [wi["w"][in_sz:], wo["w"][in_sz:], wg["w"][in_sz:]], axis=1)
    wf_h = wf["w"][in_sz:]
    return w_all, b_all, wh_all, wf_h


def _run_lstm(a, bgraph_flat, wh_all, wf_h, depth, nk, be, bpe=None):
    h, hc = _lstm_init(a, be, bpe)
    for it in range(depth - 1):
        ghc = _gather_rows(hc, bgraph_flat)
        h, hc = _lstm_step(a, ghc, wh_all, wf_h, nk, be, it == depth - 2, bpe)
    return h, hc


def _tree_encoder(enc, hx, f0, pos1h, agraph_flat, bgraph_flat, depth, nk, be_e, be_n):
    in_sz = _HID + pos1h.shape[1]
    in_pad = 384
    w_all, b_all, wh_all, wf_h = _pack_lstm(enc["lstm"], in_sz, in_pad)
    hx_src = _gather_rows(hx, f0)
    x = jnp.concatenate(
        [hx_src, pos1h,
         jnp.zeros((hx_src.shape[0], in_pad - in_sz), jnp.float32)], axis=1)
    a = _dense(x, w_all, b_all, None, False, be_e)
    h, hpk = _run_lstm(a, bgraph_flat, wh_all, wf_h, depth, nk, be_e)
    gn = _gather_rows(hpk, agraph_flat)
    nh = _sumcat_dense(hx, gn, enc["Wo"]["w"], enc["Wo"]["b"], nk, "relu", True,
                       be_n, packed_g=True)
    return nh, h


def kernel(tree_fnode, tree_fmess, tree_agraph, tree_bgraph, tree_cgraph,
           roots, graph_fnode, graph_fmess, graph_agraph, graph_bgraph, params):
    p = params
    depth = 3
    nei_g, nei_t, cw = 6, 6, 8

    genc = p["graph_encoder"]
    in_sz_g = 40 + 4 + 20
    w_all_g, b_all_g, wh_all_g, wf_h_g = _pack_lstm(genc["lstm"], in_sz_g, 128)
    src_atom = jnp.take(graph_fnode, graph_fmess[:, 0], axis=0)
    xg = jnp.concatenate(
        [jax.nn.one_hot(src_atom, 40, dtype=jnp.float32),
         jax.nn.one_hot(graph_fmess[:, 2], 4, dtype=jnp.float32),
         jax.nn.one_hot(graph_fmess[:, 3], 20, dtype=jnp.float32),
         jnp.zeros((graph_fmess.shape[0], 128 - in_sz_g), jnp.float32)], axis=1)
    a_g = _dense(xg, w_all_g, b_all_g, None, False, 1000)
    _, hpk_g = _run_lstm(a_g, graph_bgraph.reshape(-1), wh_all_g, wf_h_g,
                         depth, nei_g, 1000)
    gn_g = _gather_rows(hpk_g, graph_agraph.reshape(-1))
    fnode_g = jnp.pad(jax.nn.one_hot(graph_fnode, 40, dtype=jnp.float32),
                      ((0, 0), (0, 88)))
    wo_g = jnp.concatenate(
        [jnp.pad(genc["Wo"]["w"][:40], ((0, 88), (0, 0))), genc["Wo"]["w"][40:]],
        axis=0)
    hatom, hatom_pk = _sumcat_dense(fnode_g, gn_g, wo_g, genc["Wo"]["b"], nei_g,
                                    "relu", True, 1000, packed_g=True,
                                    emit_pk=True)

    f0 = tree_fmess[:, 0]
    pos1h = jax.nn.one_hot(tree_fmess[:, 2], 20, dtype=jnp.float32)
    ag_flat = tree_agraph.reshape(-1)
    bg_flat = tree_bgraph.reshape(-1)

    emb_tab = jnp.concatenate([p["E_l"], p["E_i"], p["E_c"]], axis=0)
    nl, ni = p["E_l"].shape[0], p["E_i"].shape[0]
    emb_idx = jnp.concatenate(
        [tree_fnode[:, 2], nl + tree_fnode[:, 1], nl + ni + tree_fnode[:, 0]])
    nt = tree_fnode.shape[0]
    emb = _gather_rows(emb_tab, emb_idx)
    hnode_b, finput1, finput = emb[:nt], emb[nt:2 * nt], emb[2 * nt:3 * nt]

    g_cg = _gather_rows(hatom_pk, tree_cgraph.reshape(-1))
    hn = _sumcat_dense(finput1, g_cg, p["W_i"]["w"], p["W_i"]["b"], cw,
                       "relu", False, 1000, packed_g=True)

    et = tree_fmess.shape[0]
    in_sz_t = _HID + 20
    pk_b = _pack_lstm(p["bond_encoder"]["lstm"], in_sz_t, 384)
    pk_f = _pack_lstm(p["frag_encoder"]["lstm"], in_sz_t, 384)
    w_all2 = jnp.concatenate([pk_b[0], pk_f[0]], axis=0)
    b_all2 = jnp.concatenate([pk_b[1], pk_f[1]])
    wh2 = jnp.concatenate([pk_b[2], pk_f[2]], axis=0)
    wfh2 = jnp.concatenate([pk_b[3], pk_f[3]], axis=0)
    hx2 = jnp.concatenate([hnode_b, hn], axis=0)
    f0_2 = jnp.concatenate([f0, nt + f0])
    hx_src2 = _gather_rows(hx2, f0_2)
    pos2 = jnp.concatenate([pos1h, pos1h], axis=0)
    x2 = jnp.concatenate(
        [hx_src2, pos2, jnp.zeros((2 * et, 384 - in_sz_t), jnp.float32)], axis=1)
    a2 = _dense(x2, w_all2, b_all2, None, False, 1000, bpe=et // 1000)
    bg2 = jnp.concatenate([bg_flat, et + bg_flat])
    _, hpk2 = _run_lstm(a2, bg2, wh2, wfh2, depth, nei_t, 1000, bpe=et // 1000)
    ag2 = jnp.concatenate([ag_flat, et + ag_flat])
    gn2 = _gather_rows(hpk2, ag2)
    wo2 = jnp.concatenate([p["bond_encoder"]["Wo"]["w"],
                           p["frag_encoder"]["Wo"]["w"]], axis=0)
    bo2 = jnp.concatenate([p["bond_encoder"]["Wo"]["b"],
                           p["frag_encoder"]["Wo"]["b"]])
    nh2 = _sumcat_dense(hx2, gn2, wo2, bo2, nei_t, "relu", True, 1000,
                        packed_g=True, bpe=nt // 1000)
    hbond, hinter = nh2[:nt], nh2[nt:]

    xc = jnp.concatenate([finput, hinter, hbond], axis=1)
    hnode_i = _dense(xc, p["W_c"]["w"], p["W_c"]["b"], "relu", False, 1000)
    hnode, hmess = _tree_encoder(p["inter_encoder"], hnode_i, f0, pos1h,
                                 ag_flat, bg_flat, depth, nei_t, 1000, 1000)

    agr = jnp.take(tree_agraph, roots, axis=0).reshape(-1)
    root_tab = jnp.concatenate([hnode_i, hmess], axis=0)
    root_idx = jnp.concatenate([roots, hnode_i.shape[0] + agr])
    rg = _gather_rows(root_tab, root_idx, ch=8)
    nr = roots.shape[0]
    fnode_r, g_r = rg[:nr], rg[nr:nr + agr.shape[0]]
    hroot = _sumcat_dense(fnode_r, g_r, p["W_root"]["w"], p["W_root"]["b"],
                          nei_t, "tanh", False, 128)

    return hroot, hnode, hinter, hbond, hatom


# speedup vs baseline: 2.3851x; 1.1089x over previous
import functools

import jax
import jax.numpy as jnp
from jax import lax
from jax.experimental import pallas as pl
from jax.experimental.pallas import tpu as pltpu
from jax.experimental.pallas import tpu_sc as plsc

_HID = 256
_NC = 2
_NS = 16
_NW = _NC * _NS


def _gather_rows(table, idx, ch=128, trim=True):
    n = idx.shape[0]
    d = table.shape[1]
    dt = table.dtype
    unit = _NW * ch
    n_pad = ((n + unit - 1) // unit) * unit
    if n_pad != n:
        idx = jnp.pad(idx, (0, n_pad - n))
    chunks = n_pad // unit

    pairs = chunks // 2
    tail = chunks % 2

    def body(tab_ref, idx_ref, out_ref, i0, i1, r0, r1, si0, si1, sg, sg1, sw0, sw1):
        wid = lax.axis_index("s") * _NC + lax.axis_index("c")
        base = wid * chunks

        def idx_cp(t, iv, sem):
            return pltpu.make_async_copy(idx_ref.at[pl.ds((base + t) * ch, ch)],
                                         iv, sem)

        def wb_cp(t, rv, sem):
            return pltpu.make_async_copy(rv, out_ref.at[pl.ds((base + t) * ch, ch)],
                                         sem)

        idx_cp(0, i0, si0).start()
        if chunks > 1:
            idx_cp(1, i1, si1).start()

        def g_cp(iv, rv, sem):
            return pltpu.make_async_copy(tab_ref.at[iv], rv, sem)

        def fire(t, iv, siv, rv, swv, sgv, p):
            idx_cp(t, iv, siv).wait()

            @pl.when(p > 0)
            def _():
                wb_cp(t, rv, swv).wait()

            g_cp(iv, rv, sgv).start()

        def drain(t, iv, siv, rv, swv, sgv):
            g_cp(iv, rv, sgv).wait()
            wb_cp(t, rv, swv).start()

            @pl.when(t + 2 < chunks)
            def _():
                idx_cp(t + 2, iv, siv).start()

        def pair(p, carry):
            t0 = jnp.int32(2 * p)
            t1 = t0 + 1
            fire(t0, i0, si0, r0, sw0, sg, p)
            fire(t1, i1, si1, r1, sw1, sg1, p)
            drain(t0, i0, si0, r0, sw0, sg)
            drain(t1, i1, si1, r1, sw1, sg1)
            return carry

        if pairs > 0:
            lax.fori_loop(0, pairs, pair, 0)
        if tail:
            t = jnp.int32(chunks - 1)
            fire(t, i0, si0, r0, sw0, sg, jnp.int32(pairs))
            drain(t, i0, si0, r0, sw0, sg)
        last = chunks - 1
        if chunks >= 2:
            prev = last - 1
            wb_cp(prev, r0 if prev % 2 == 0 else r1,
                  sw0 if prev % 2 == 0 else sw1).wait()
        wb_cp(last, r0 if last % 2 == 0 else r1,
              sw0 if last % 2 == 0 else sw1).wait()

    mesh = plsc.VectorSubcoreMesh(core_axis_name="c", subcore_axis_name="s")
    out = pl.kernel(
        body,
        mesh=mesh,
        out_type=jax.ShapeDtypeStruct((n_pad, d), dt),
        scratch_types=[
            pltpu.VMEM((ch,), jnp.int32),
            pltpu.VMEM((ch,), jnp.int32),
            pltpu.VMEM((ch, d), dt),
            pltpu.VMEM((ch, d), dt),
            pltpu.SemaphoreType.DMA,
            pltpu.SemaphoreType.DMA,
            pltpu.SemaphoreType.DMA,
            pltpu.SemaphoreType.DMA,
            pltpu.SemaphoreType.DMA,
            pltpu.SemaphoreType.DMA,
        ],
    )(table, idx)
    return out[:n] if trim else out


def _apply_act(y, act):
    if act == "relu":
        return jnp.maximum(y, 0.0)
    if act == "tanh":
        return jnp.tanh(y)
    return y


def _mask0(y, i_blk, be, bpe):
    row = (i_blk % bpe) * be + lax.broadcasted_iota(jnp.int32, (be, 1), 0)
    return jnp.where(row == 0, 0.0, y)


def _dense(x, w, b, act, mask0, be, bpe=None):
    n, k = x.shape
    m = w.shape[1]
    if bpe is None:
        bpe = n // be
        w = w.reshape(1, k, m)
        b2 = b.reshape(1, 1, m)
    else:
        w = w.reshape(-1, k, m)
        b2 = b.reshape(-1, 1, m)

    def body(x_ref, w_ref, b_ref, o_ref):
        y = jnp.dot(x_ref[...], w_ref[0], preferred_element_type=jnp.float32)
        y = y + b_ref[0]
        y = _apply_act(y, act)
        if mask0:
            y = _mask0(y, pl.program_id(0), be, bpe)
        o_ref[...] = y

    return pl.pallas_call(
        body,
        grid=(n // be,),
        in_specs=[
            pl.BlockSpec((be, k), lambda i: (i, 0)),
            pl.BlockSpec((1, k, m), lambda i: (i // bpe, 0, 0)),
            pl.BlockSpec((1, 1, m), lambda i: (i // bpe, 0, 0)),
        ],
        out_specs=pl.BlockSpec((be, m), lambda i: (i, 0)),
        out_shape=jax.ShapeDtypeStruct((n, m), jnp.float32),
    )(x, w, b2)


def _pack_hc(h, c):
    hu = lax.bitcast_convert_type(h, jnp.uint32)
    cu = lax.bitcast_convert_type(c, jnp.uint32)
    pk = ((hu + 0x8000) & jnp.uint32(0xFFFF0000)) | ((cu + 0x8000) >> 16)
    return lax.bitcast_convert_type(pk, jnp.int32)


def _unpack_hc(g):
    gu = lax.bitcast_convert_type(g, jnp.uint32)
    hv = lax.bitcast_convert_type(gu & jnp.uint32(0xFFFF0000), jnp.float32)
    cv = lax.bitcast_convert_type(gu << 16, jnp.float32)
    return hv, cv


def _pack_cols(y):
    return _pack_hc(y[:, :_HID // 2], y[:, _HID // 2:])


def _unpack_cols(g):
    y1, y2 = _unpack_hc(g)
    return jnp.concatenate([y1, y2], axis=1)


def _lstm_init(a, be, bpe=None):
    e = a.shape[0]
    h = _HID
    if bpe is None:
        bpe = e // be

    def body(a_ref, h_ref, hc_ref):
        av = a_ref[...]
        gi = jax.nn.sigmoid(av[:, 0 * h:1 * h])
        go = jax.nn.sigmoid(av[:, 1 * h:2 * h])
        gg = jnp.tanh(av[:, 3 * h:4 * h])
        c_new = gi * gg
        h_new = go * jnp.tanh(c_new)
        h_new = _mask0(h_new, pl.program_id(0), be, bpe)
        c_new = _mask0(c_new, pl.program_id(0), be, bpe)
        h_ref[...] = h_new
        hc_ref[...] = _pack_hc(h_new, c_new)

    return pl.pallas_call(
        body,
        grid=(e // be,),
        in_specs=[pl.BlockSpec((be, 4 * h), lambda i: (i, 0))],
        out_specs=[
            pl.BlockSpec((be, h), lambda i: (i, 0)),
            pl.BlockSpec((be, h), lambda i: (i, 0)),
        ],
        out_shape=[
            jax.ShapeDtypeStruct((e, h), jnp.float32),
            jax.ShapeDtypeStruct((e, h), jnp.int32),
        ],
    )(a)


def _lstm_step(a, ghc, wh_all, wf_h, nk, be, last, bpe=None):
    e = a.shape[0]
    h = _HID
    if bpe is None:
        bpe = e // be
    wh_all = wh_all.reshape(-1, h, 3 * h)
    wf_h = wf_h.reshape(-1, h, h)

    def body(a_ref, g_ref, wh_ref, wf_ref, h_ref, p_ref):
        av = a_ref[...]
        ghv, gcv = _unpack_hc(g_ref[...])
        gcv = gcv.reshape(be, nk, h)
        hs = ghv.reshape(be, nk, h).sum(axis=1)
        zh = jnp.dot(hs, wh_ref[0], preferred_element_type=jnp.float32)
        gi = jax.nn.sigmoid(av[:, 0 * h:1 * h] + zh[:, 0 * h:1 * h])
        go = jax.nn.sigmoid(av[:, 1 * h:2 * h] + zh[:, 1 * h:2 * h])
        gg = jnp.tanh(av[:, 3 * h:4 * h] + zh[:, 2 * h:3 * h])
        fpre = jnp.dot(ghv, wf_ref[0], preferred_element_type=jnp.float32)
        gf = jax.nn.sigmoid(fpre.reshape(be, nk, h) + av[:, 2 * h:3 * h][:, None, :])
        c_new = gi * gg + (gf * gcv).sum(axis=1)
        h_new = go * jnp.tanh(c_new)
        h_new = _mask0(h_new, pl.program_id(0), be, bpe)
        c_new = _mask0(c_new, pl.program_id(0), be, bpe)
        h_ref[...] = h_new
        if last:
            p_ref[...] = _pack_cols(h_new)
        else:
            p_ref[...] = _pack_hc(h_new, c_new)

    pw = h // 2 if last else h
    return pl.pallas_call(
        body,
        grid=(e // be,),
        in_specs=[
            pl.BlockSpec((be, 4 * h), lambda i: (i, 0)),
            pl.BlockSpec((be * nk, h), lambda i: (i, 0)),
            pl.BlockSpec((1, h, 3 * h), lambda i: (i // bpe, 0, 0)),
            pl.BlockSpec((1, h, h), lambda i: (i // bpe, 0, 0)),
        ],
        out_specs=[
            pl.BlockSpec((be, h), lambda i: (i, 0)),
            pl.BlockSpec((be, pw), lambda i: (i, 0)),
        ],
        out_shape=[
            jax.ShapeDtypeStruct((e, h), jnp.float32),
            jax.ShapeDtypeStruct((e, pw), jnp.int32),
        ],
    )(a, ghc, wh_all, wf_h)


def _sumcat_dense(x1, g, w, b, nk, act, mask0, be, packed_g=False, emit_pk=False,
                  bpe=None):
    n, d1 = x1.shape
    h = _HID
    m = w.shape[1]
    if bpe is None:
        bpe = n // be
    w = w.reshape(-1, d1 + h, m)
    b2 = b.reshape(-1, 1, m)
    gw = h // 2 if packed_g else h

    def body(x_ref, g_ref, w_ref, b_ref, *o_refs):
        gv = g_ref[...]
        if packed_g:
            gv = _unpack_cols(gv)
        nei = gv.reshape(be, nk, h).sum(axis=1)
        xcat = jnp.concatenate([x_ref[...], nei], axis=1)
        y = jnp.dot(xcat, w_ref[0], preferred_element_type=jnp.float32)
        y = y + b_ref[0]
        y = _apply_act(y, act)
        if mask0:
            y = _mask0(y, pl.program_id(0), be, bpe)
        o_refs[0][...] = y
        if emit_pk:
            o_refs[1][...] = _pack_cols(y)

    out_specs = [pl.BlockSpec((be, m), lambda i: (i, 0))]
    out_shape = [jax.ShapeDtypeStruct((n, m), jnp.float32)]
    if emit_pk:
        out_specs.append(pl.BlockSpec((be, m // 2), lambda i: (i, 0)))
        out_shape.append(jax.ShapeDtypeStruct((n, m // 2), jnp.int32))
    res = pl.pallas_call(
        body,
        grid=(n // be,),
        in_specs=[
            pl.BlockSpec((be, d1), lambda i: (i, 0)),
            pl.BlockSpec((be * nk, gw), lambda i: (i, 0)),
            pl.BlockSpec((1, d1 + h, m), lambda i: (i // bpe, 0, 0)),
            pl.BlockSpec((1, 1, m), lambda i: (i // bpe, 0, 0)),
        ],
        out_specs=out_specs,
        out_shape=out_shape,
    )(x1, g, w, b2)
    return res if emit_pk else res[0]


def _pack_lstm(lstm, in_sz, in_pad):
    wi, wo, wf, wg = lstm["Wi"], lstm["Wog"], lstm["Wf"], lstm["W"]
    w_all = jnp.concatenate(
        [wi["w"][:in_sz], wo["w"][:in_sz], wf["w"][:in_sz], wg["w"][:in_sz]], axis=1)
    if in_pad != in_sz:
        w_all = jnp.pad(w_all, ((0, in_pad - in_sz), (0, 0)))
    b_all = jnp.concatenate([wi["b"], wo["b"], wf["b"], wg["b"]])
    wh_all = jnp.concatenate(
        [wi["w"][in_sz:], wo["w"][in_sz:], wg["w"][in_sz:]], axis=1)
    wf_h = wf["w"][in_sz:]
    return w_all, b_all, wh_all, wf_h


def _run_lstm(a, bgraph_flat, wh_all, wf_h, depth, nk, be, bpe=None):
    h, hc = _lstm_init(a, be, bpe)
    for it in range(depth - 1):
        ghc = _gather_rows(hc, bgraph_flat, trim=False)
        h, hc = _lstm_step(a, ghc, wh_all, wf_h, nk, be, it == depth - 2, bpe)
    return h, hc


def _tree_encoder(enc, hx, f0, pos1h, agraph_flat, bgraph_flat, depth, nk, be_e, be_n):
    in_sz = _HID + pos1h.shape[1]
    in_pad = 384
    w_all, b_all, wh_all, wf_h = _pack_lstm(enc["lstm"], in_sz, in_pad)
    hx_src = _gather_rows(hx, f0)
    x = jnp.concatenate(
        [hx_src, pos1h,
         jnp.zeros((hx_src.shape[0], in_pad - in_sz), jnp.float32)], axis=1)
    a = _dense(x, w_all, b_all, None, False, be_e)
    h, hpk = _run_lstm(a, bgraph_flat, wh_all, wf_h, depth, nk, be_e)
    gn = _gather_rows(hpk, agraph_flat, trim=False)
    nh = _sumcat_dense(hx, gn, enc["Wo"]["w"], enc["Wo"]["b"], nk, "relu", True,
                       be_n, packed_g=True)
    return nh, h


def kernel(tree_fnode, tree_fmess, tree_agraph, tree_bgraph, tree_cgraph,
           roots, graph_fnode, graph_fmess, graph_agraph, graph_bgraph, params):
    p = params
    depth = 3
    nei_g, nei_t, cw = 6, 6, 8

    genc = p["graph_encoder"]
    in_sz_g = 40 + 4 + 20
    w_all_g, b_all_g, wh_all_g, wf_h_g = _pack_lstm(genc["lstm"], in_sz_g, 128)
    src_atom = jnp.take(graph_fnode, graph_fmess[:, 0], axis=0)
    xg = jnp.concatenate(
        [jax.nn.one_hot(src_atom, 40, dtype=jnp.float32),
         jax.nn.one_hot(graph_fmess[:, 2], 4, dtype=jnp.float32),
         jax.nn.one_hot(graph_fmess[:, 3], 20, dtype=jnp.float32),
         jnp.zeros((graph_fmess.shape[0], 128 - in_sz_g), jnp.float32)], axis=1)
    a_g = _dense(xg, w_all_g, b_all_g, None, False, 1000)
    _, hpk_g = _run_lstm(a_g, graph_bgraph.reshape(-1), wh_all_g, wf_h_g,
                         depth, nei_g, 1000)
    gn_g = _gather_rows(hpk_g, graph_agraph.reshape(-1), trim=False)
    fnode_g = jnp.pad(jax.nn.one_hot(graph_fnode, 40, dtype=jnp.float32),
                      ((0, 0), (0, 88)))
    wo_g = jnp.concatenate(
        [jnp.pad(genc["Wo"]["w"][:40], ((0, 88), (0, 0))), genc["Wo"]["w"][40:]],
        axis=0)
    hatom, hatom_pk = _sumcat_dense(fnode_g, gn_g, wo_g, genc["Wo"]["b"], nei_g,
                                    "relu", True, 1000, packed_g=True,
                                    emit_pk=True)

    f0 = tree_fmess[:, 0]
    pos1h = jax.nn.one_hot(tree_fmess[:, 2], 20, dtype=jnp.float32)
    ag_flat = tree_agraph.reshape(-1)
    bg_flat = tree_bgraph.reshape(-1)

    emb_tab = jnp.concatenate([p["E_l"], p["E_i"], p["E_c"]], axis=0)
    nl, ni = p["E_l"].shape[0], p["E_i"].shape[0]
    emb_idx = jnp.concatenate(
        [tree_fnode[:, 2], nl + tree_fnode[:, 1], nl + ni + tree_fnode[:, 0]])
    nt = tree_fnode.shape[0]
    emb = _gather_rows(emb_tab, emb_idx)
    hnode_b, finput1, finput = emb[:nt], emb[nt:2 * nt], emb[2 * nt:3 * nt]

    g_cg = _gather_rows(hatom_pk, tree_cgraph.reshape(-1), trim=False)
    hn = _sumcat_dense(finput1, g_cg, p["W_i"]["w"], p["W_i"]["b"], cw,
                       "relu", False, 1000, packed_g=True)

    et = tree_fmess.shape[0]
    in_sz_t = _HID + 20
    pk_b = _pack_lstm(p["bond_encoder"]["lstm"], in_sz_t, 384)
    pk_f = _pack_lstm(p["frag_encoder"]["lstm"], in_sz_t, 384)
    w_all2 = jnp.concatenate([pk_b[0], pk_f[0]], axis=0)
    b_all2 = jnp.concatenate([pk_b[1], pk_f[1]])
    wh2 = jnp.concatenate([pk_b[2], pk_f[2]], axis=0)
    wfh2 = jnp.concatenate([pk_b[3], pk_f[3]], axis=0)
    hx2 = jnp.concatenate([hnode_b, hn], axis=0)
    f0_2 = jnp.concatenate([f0, nt + f0])
    hx_src2 = _gather_rows(hx2, f0_2)
    pos2 = jnp.concatenate([pos1h, pos1h], axis=0)
    x2 = jnp.concatenate(
        [hx_src2, pos2, jnp.zeros((2 * et, 384 - in_sz_t), jnp.float32)], axis=1)
    a2 = _dense(x2, w_all2, b_all2, None, False, 1000, bpe=et // 1000)
    bg2 = jnp.concatenate([bg_flat, et + bg_flat])
    _, hpk2 = _run_lstm(a2, bg2, wh2, wfh2, depth, nei_t, 1000, bpe=et // 1000)
    ag2 = jnp.concatenate([ag_flat, et + ag_flat])
    gn2 = _gather_rows(hpk2, ag2, trim=False)
    wo2 = jnp.concatenate([p["bond_encoder"]["Wo"]["w"],
                           p["frag_encoder"]["Wo"]["w"]], axis=0)
    bo2 = jnp.concatenate([p["bond_encoder"]["Wo"]["b"],
                           p["frag_encoder"]["Wo"]["b"]])
    nh2 = _sumcat_dense(hx2, gn2, wo2, bo2, nei_t, "relu", True, 1000,
                        packed_g=True, bpe=nt // 1000)
    hbond, hinter = nh2[:nt], nh2[nt:]

    xc = jnp.concatenate([finput, hinter, hbond], axis=1)
    hnode_i = _dense(xc, p["W_c"]["w"], p["W_c"]["b"], "relu", False, 1000)
    hnode, hmess = _tree_encoder(p["inter_encoder"], hnode_i, f0, pos1h,
                                 ag_flat, bg_flat, depth, nei_t, 1000, 1000)

    agr = jnp.take(tree_agraph, roots, axis=0).reshape(-1)
    root_tab = jnp.concatenate([hnode_i, hmess], axis=0)
    root_idx = jnp.concatenate([roots, hnode_i.shape[0] + agr])
    rg = _gather_rows(root_tab, root_idx, ch=8)
    nr = roots.shape[0]
    fnode_r, g_r = rg[:nr], rg[nr:nr + agr.shape[0]]
    hroot = _sumcat_dense(fnode_r, g_r, p["W_root"]["w"], p["W_root"]["b"],
                          nei_t, "tanh", False, 128)

    return hroot, hnode, hinter, hbond, hatom


# speedup vs baseline: 2.4150x; 1.0126x over previous
import functools

import jax
import jax.numpy as jnp
from jax import lax
from jax.experimental import pallas as pl
from jax.experimental.pallas import tpu as pltpu
from jax.experimental.pallas import tpu_sc as plsc

_HID = 256
_NC = 2
_NS = 16
_NW = _NC * _NS


def _gather_rows(table, idx, ch=128, trim=True):
    n = idx.shape[0]
    d = table.shape[1]
    dt = table.dtype
    unit = _NW * ch
    n_pad = ((n + unit - 1) // unit) * unit
    if n_pad != n:
        idx = jnp.pad(idx, (0, n_pad - n))
    chunks = n_pad // unit

    pairs = chunks // 2
    tail = chunks % 2

    def body(tab_ref, idx_ref, out_ref, i0, i1, r0, r1, si0, si1, sg, sg1, sw0, sw1):
        wid = lax.axis_index("s") * _NC + lax.axis_index("c")
        base = wid * chunks

        def idx_cp(t, iv, sem):
            return pltpu.make_async_copy(idx_ref.at[pl.ds((base + t) * ch, ch)],
                                         iv, sem)

        def wb_cp(t, rv, sem):
            return pltpu.make_async_copy(rv, out_ref.at[pl.ds((base + t) * ch, ch)],
                                         sem)

        idx_cp(0, i0, si0).start()
        if chunks > 1:
            idx_cp(1, i1, si1).start()

        def g_cp(iv, rv, sem):
            return pltpu.make_async_copy(tab_ref.at[iv], rv, sem)

        def fire(t, iv, siv, rv, swv, sgv, p):
            idx_cp(t, iv, siv).wait()

            @pl.when(p > 0)
            def _():
                wb_cp(t, rv, swv).wait()

            g_cp(iv, rv, sgv).start()

        def drain(t, iv, siv, rv, swv, sgv):
            g_cp(iv, rv, sgv).wait()
            wb_cp(t, rv, swv).start()

            @pl.when(t + 2 < chunks)
            def _():
                idx_cp(t + 2, iv, siv).start()

        def pair(p, carry):
            t0 = jnp.int32(2 * p)
            t1 = t0 + 1
            fire(t0, i0, si0, r0, sw0, sg, p)
            fire(t1, i1, si1, r1, sw1, sg1, p)
            drain(t0, i0, si0, r0, sw0, sg)
            drain(t1, i1, si1, r1, sw1, sg1)
            return carry

        if pairs > 0:
            lax.fori_loop(0, pairs, pair, 0)
        if tail:
            t = jnp.int32(chunks - 1)
            fire(t, i0, si0, r0, sw0, sg, jnp.int32(pairs))
            drain(t, i0, si0, r0, sw0, sg)
        last = chunks - 1
        if chunks >= 2:
            prev = last - 1
            wb_cp(prev, r0 if prev % 2 == 0 else r1,
                  sw0 if prev % 2 == 0 else sw1).wait()
        wb_cp(last, r0 if last % 2 == 0 else r1,
              sw0 if last % 2 == 0 else sw1).wait()

    mesh = plsc.VectorSubcoreMesh(core_axis_name="c", subcore_axis_name="s")
    out = pl.kernel(
        body,
        mesh=mesh,
        out_type=jax.ShapeDtypeStruct((n_pad, d), dt),
        scratch_types=[
            pltpu.VMEM((ch,), jnp.int32),
            pltpu.VMEM((ch,), jnp.int32),
            pltpu.VMEM((ch, d), dt),
            pltpu.VMEM((ch, d), dt),
            pltpu.SemaphoreType.DMA,
            pltpu.SemaphoreType.DMA,
            pltpu.SemaphoreType.DMA,
            pltpu.SemaphoreType.DMA,
            pltpu.SemaphoreType.DMA,
            pltpu.SemaphoreType.DMA,
        ],
    )(table, idx)
    return out[:n] if trim else out


def _bdot(x, w):
    return jnp.dot(x.astype(jnp.bfloat16), w.astype(jnp.bfloat16),
                   preferred_element_type=jnp.float32)


def _apply_act(y, act):
    if act == "relu":
        return jnp.maximum(y, 0.0)
    if act == "tanh":
        return jnp.tanh(y)
    return y


def _mask0(y, i_blk, be, bpe):
    row = (i_blk % bpe) * be + lax.broadcasted_iota(jnp.int32, (be, 1), 0)
    return jnp.where(row == 0, 0.0, y)


def _dense(x, w, b, act, mask0, be, bpe=None):
    n, k = x.shape
    m = w.shape[1]
    if bpe is None:
        bpe = n // be
        w = w.reshape(1, k, m)
        b2 = b.reshape(1, 1, m)
    else:
        w = w.reshape(-1, k, m)
        b2 = b.reshape(-1, 1, m)

    def body(x_ref, w_ref, b_ref, o_ref):
        y = jnp.dot(x_ref[...], w_ref[0], preferred_element_type=jnp.float32)
        y = y + b_ref[0]
        y = _apply_act(y, act)
        if mask0:
            y = _mask0(y, pl.program_id(0), be, bpe)
        o_ref[...] = y

    return pl.pallas_call(
        body,
        grid=(n // be,),
        in_specs=[
            pl.BlockSpec((be, k), lambda i: (i, 0)),
            pl.BlockSpec((1, k, m), lambda i: (i // bpe, 0, 0)),
            pl.BlockSpec((1, 1, m), lambda i: (i // bpe, 0, 0)),
        ],
        out_specs=pl.BlockSpec((be, m), lambda i: (i, 0)),
        out_shape=jax.ShapeDtypeStruct((n, m), jnp.float32),
    )(x, w, b2)


def _pack_hc(h, c):
    hu = lax.bitcast_convert_type(h, jnp.uint32)
    cu = lax.bitcast_convert_type(c, jnp.uint32)
    pk = ((hu + 0x8000) & jnp.uint32(0xFFFF0000)) | ((cu + 0x8000) >> 16)
    return lax.bitcast_convert_type(pk, jnp.int32)


def _unpack_hc(g):
    gu = lax.bitcast_convert_type(g, jnp.uint32)
    hv = lax.bitcast_convert_type(gu & jnp.uint32(0xFFFF0000), jnp.float32)
    cv = lax.bitcast_convert_type(gu << 16, jnp.float32)
    return hv, cv


def _pack_cols(y):
    return _pack_hc(y[:, :_HID // 2], y[:, _HID // 2:])


def _unpack_cols(g):
    y1, y2 = _unpack_hc(g)
    return jnp.concatenate([y1, y2], axis=1)


def _lstm_init(a, be, bpe=None):
    e = a.shape[0]
    h = _HID
    if bpe is None:
        bpe = e // be

    def body(a_ref, h_ref, hc_ref):
        av = a_ref[...]
        gi = jax.nn.sigmoid(av[:, 0 * h:1 * h])
        go = jax.nn.sigmoid(av[:, 1 * h:2 * h])
        gg = jnp.tanh(av[:, 3 * h:4 * h])
        c_new = gi * gg
        h_new = go * jnp.tanh(c_new)
        h_new = _mask0(h_new, pl.program_id(0), be, bpe)
        c_new = _mask0(c_new, pl.program_id(0), be, bpe)
        h_ref[...] = h_new
        hc_ref[...] = _pack_hc(h_new, c_new)

    return pl.pallas_call(
        body,
        grid=(e // be,),
        in_specs=[pl.BlockSpec((be, 4 * h), lambda i: (i, 0))],
        out_specs=[
            pl.BlockSpec((be, h), lambda i: (i, 0)),
            pl.BlockSpec((be, h), lambda i: (i, 0)),
        ],
        out_shape=[
            jax.ShapeDtypeStruct((e, h), jnp.float32),
            jax.ShapeDtypeStruct((e, h), jnp.int32),
        ],
    )(a)


def _lstm_step(a, ghc, wh_all, wf_h, nk, be, last, bpe=None):
    e = a.shape[0]
    h = _HID
    if bpe is None:
        bpe = e // be
    wh_all = wh_all.reshape(-1, h, 3 * h)
    wf_h = wf_h.reshape(-1, h, h)

    def body(a_ref, g_ref, wh_ref, wf_ref, h_ref, p_ref):
        av = a_ref[...]
        ghv, gcv = _unpack_hc(g_ref[...])
        gcv = gcv.reshape(be, nk, h)
        hs = ghv.reshape(be, nk, h).sum(axis=1)
        zh = _bdot(hs, wh_ref[0])
        gi = jax.nn.sigmoid(av[:, 0 * h:1 * h] + zh[:, 0 * h:1 * h])
        go = jax.nn.sigmoid(av[:, 1 * h:2 * h] + zh[:, 1 * h:2 * h])
        gg = jnp.tanh(av[:, 3 * h:4 * h] + zh[:, 2 * h:3 * h])
        fpre = _bdot(ghv, wf_ref[0])
        gf = jax.nn.sigmoid(fpre.reshape(be, nk, h) + av[:, 2 * h:3 * h][:, None, :])
        c_new = gi * gg + (gf * gcv).sum(axis=1)
        h_new = go * jnp.tanh(c_new)
        h_new = _mask0(h_new, pl.program_id(0), be, bpe)
        c_new = _mask0(c_new, pl.program_id(0), be, bpe)
        h_ref[...] = h_new
        if last:
            p_ref[...] = _pack_cols(h_new)
        else:
            p_ref[...] = _pack_hc(h_new, c_new)

    pw = h // 2 if last else h
    return pl.pallas_call(
        body,
        grid=(e // be,),
        in_specs=[
            pl.BlockSpec((be, 4 * h), lambda i: (i, 0)),
            pl.BlockSpec((be * nk, h), lambda i: (i, 0)),
            pl.BlockSpec((1, h, 3 * h), lambda i: (i // bpe, 0, 0)),
            pl.BlockSpec((1, h, h), lambda i: (i // bpe, 0, 0)),
        ],
        out_specs=[
            pl.BlockSpec((be, h), lambda i: (i, 0)),
            pl.BlockSpec((be, pw), lambda i: (i, 0)),
        ],
        out_shape=[
            jax.ShapeDtypeStruct((e, h), jnp.float32),
            jax.ShapeDtypeStruct((e, pw), jnp.int32),
        ],
    )(a, ghc, wh_all, wf_h)


def _sumcat_dense(x1, g, w, b, nk, act, mask0, be, packed_g=False, emit_pk=False,
                  bpe=None):
    n, d1 = x1.shape
    h = _HID
    m = w.shape[1]
    if bpe is None:
        bpe = n // be
    w = w.reshape(-1, d1 + h, m)
    b2 = b.reshape(-1, 1, m)
    gw = h // 2 if packed_g else h

    def body(x_ref, g_ref, w_ref, b_ref, *o_refs):
        gv = g_ref[...]
        if packed_g:
            gv = _unpack_cols(gv)
        nei = gv.reshape(be, nk, h).sum(axis=1)
        xcat = jnp.concatenate([x_ref[...], nei], axis=1)
        y = _bdot(xcat, w_ref[0])
        y = y + b_ref[0]
        y = _apply_act(y, act)
        if mask0:
            y = _mask0(y, pl.program_id(0), be, bpe)
        o_refs[0][...] = y
        if emit_pk:
            o_refs[1][...] = _pack_cols(y)

    out_specs = [pl.BlockSpec((be, m), lambda i: (i, 0))]
    out_shape = [jax.ShapeDtypeStruct((n, m), jnp.float32)]
    if emit_pk:
        out_specs.append(pl.BlockSpec((be, m // 2), lambda i: (i, 0)))
        out_shape.append(jax.ShapeDtypeStruct((n, m // 2), jnp.int32))
    res = pl.pallas_call(
        body,
        grid=(n // be,),
        in_specs=[
            pl.BlockSpec((be, d1), lambda i: (i, 0)),
            pl.BlockSpec((be * nk, gw), lambda i: (i, 0)),
            pl.BlockSpec((1, d1 + h, m), lambda i: (i // bpe, 0, 0)),
            pl.BlockSpec((1, 1, m), lambda i: (i // bpe, 0, 0)),
        ],
        out_specs=out_specs,
        out_shape=out_shape,
    )(x1, g, w, b2)
    return res if emit_pk else res[0]


def _pack_lstm(lstm, in_sz, in_pad):
    wi, wo, wf, wg = lstm["Wi"], lstm["Wog"], lstm["Wf"], lstm["W"]
    w_all = jnp.concatenate(
        [wi["w"][:in_sz], wo["w"][:in_sz], wf["w"][:in_sz], wg["w"][:in_sz]], axis=1)
    if in_pad != in_sz:
        w_all = jnp.pad(w_all, ((0, in_pad - in_sz), (0, 0)))
    b_all = jnp.concatenate([wi["b"], wo["b"], wf["b"], wg["b"]])
    wh_all = jnp.concatenate(
        [wi["w"][in_sz:], wo["w"][in_sz:], wg["w"][in_sz:]], axis=1)
    wf_h = wf["w"][in_sz:]
    return w_all, b_all, wh_all, wf_h


def _run_lstm(a, bgraph_flat, wh_all, wf_h, depth, nk, be, bpe=None):
    h, hc = _lstm_init(a, be, bpe)
    for it in range(depth - 1):
        ghc = _gather_rows(hc, bgraph_flat, trim=False)
        h, hc = _lstm_step(a, ghc, wh_all, wf_h, nk, be, it == depth - 2, bpe)
    return h, hc


def _tree_encoder(enc, hx, f0, pos1h, agraph_flat, bgraph_flat, depth, nk, be_e, be_n):
    in_sz = _HID + pos1h.shape[1]
    in_pad = 384
    w_all, b_all, wh_all, wf_h = _pack_lstm(enc["lstm"], in_sz, in_pad)
    hx_src = _gather_rows(hx, f0)
    x = jnp.concatenate(
        [hx_src, pos1h,
         jnp.zeros((hx_src.shape[0], in_pad - in_sz), jnp.float32)], axis=1)
    a = _dense(x, w_all, b_all, None, False, be_e)
    h, hpk = _run_lstm(a, bgraph_flat, wh_all, wf_h, depth, nk, be_e)
    gn = _gather_rows(hpk, agraph_flat, trim=False)
    nh = _sumcat_dense(hx, gn, enc["Wo"]["w"], enc["Wo"]["b"], nk, "relu", True,
                       be_n, packed_g=True)
    return nh, h


def kernel(tree_fnode, tree_fmess, tree_agraph, tree_bgraph, tree_cgraph,
           roots, graph_fnode, graph_fmess, graph_agraph, graph_bgraph, params):
    p = params
    depth = 3
    nei_g, nei_t, cw = 6, 6, 8

    genc = p["graph_encoder"]
    in_sz_g = 40 + 4 + 20
    w_all_g, b_all_g, wh_all_g, wf_h_g = _pack_lstm(genc["lstm"], in_sz_g, 128)
    src_atom = jnp.take(graph_fnode, graph_fmess[:, 0], axis=0)
    xg = jnp.concatenate(
        [jax.nn.one_hot(src_atom, 40, dtype=jnp.float32),
         jax.nn.one_hot(graph_fmess[:, 2], 4, dtype=jnp.float32),
         jax.nn.one_hot(graph_fmess[:, 3], 20, dtype=jnp.float32),
         jnp.zeros((graph_fmess.shape[0], 128 - in_sz_g), jnp.float32)], axis=1)
    a_g = _dense(xg, w_all_g, b_all_g, None, False, 1000)
    _, hpk_g = _run_lstm(a_g, graph_bgraph.reshape(-1), wh_all_g, wf_h_g,
                         depth, nei_g, 1000)
    gn_g = _gather_rows(hpk_g, graph_agraph.reshape(-1), trim=False)
    fnode_g = jnp.pad(jax.nn.one_hot(graph_fnode, 40, dtype=jnp.float32),
                      ((0, 0), (0, 88)))
    wo_g = jnp.concatenate(
        [jnp.pad(genc["Wo"]["w"][:40], ((0, 88), (0, 0))), genc["Wo"]["w"][40:]],
        axis=0)
    hatom, hatom_pk = _sumcat_dense(fnode_g, gn_g, wo_g, genc["Wo"]["b"], nei_g,
                                    "relu", True, 1000, packed_g=True,
                                    emit_pk=True)

    f0 = tree_fmess[:, 0]
    pos1h = jax.nn.one_hot(tree_fmess[:, 2], 20, dtype=jnp.float32)
    ag_flat = tree_agraph.reshape(-1)
    bg_flat = tree_bgraph.reshape(-1)

    emb_tab = jnp.concatenate([p["E_l"], p["E_i"], p["E_c"]], axis=0)
    nl, ni = p["E_l"].shape[0], p["E_i"].shape[0]
    emb_idx = jnp.concatenate(
        [tree_fnode[:, 2], nl + tree_fnode[:, 1], nl + ni + tree_fnode[:, 0]])
    nt = tree_fnode.shape[0]
    emb = _gather_rows(emb_tab, emb_idx)
    hnode_b, finput1, finput = emb[:nt], emb[nt:2 * nt], emb[2 * nt:3 * nt]

    g_cg = _gather_rows(hatom_pk, tree_cgraph.reshape(-1), trim=False)
    hn = _sumcat_dense(finput1, g_cg, p["W_i"]["w"], p["W_i"]["b"], cw,
                       "relu", False, 1000, packed_g=True)

    et = tree_fmess.shape[0]
    in_sz_t = _HID + 20
    pk_b = _pack_lstm(p["bond_encoder"]["lstm"], in_sz_t, 384)
    pk_f = _pack_lstm(p["frag_encoder"]["lstm"], in_sz_t, 384)
    w_all2 = jnp.concatenate([pk_b[0], pk_f[0]], axis=0)
    b_all2 = jnp.concatenate([pk_b[1], pk_f[1]])
    wh2 = jnp.concatenate([pk_b[2], pk_f[2]], axis=0)
    wfh2 = jnp.concatenate([pk_b[3], pk_f[3]], axis=0)
    hx2 = jnp.concatenate([hnode_b, hn], axis=0)
    f0_2 = jnp.concatenate([f0, nt + f0])
    hx_src2 = _gather_rows(hx2, f0_2)
    pos2 = jnp.concatenate([pos1h, pos1h], axis=0)
    x2 = jnp.concatenate(
        [hx_src2, pos2, jnp.zeros((2 * et, 384 - in_sz_t), jnp.float32)], axis=1)
    a2 = _dense(x2, w_all2, b_all2, None, False, 1000, bpe=et // 1000)
    bg2 = jnp.concatenate([bg_flat, et + bg_flat])
    _, hpk2 = _run_lstm(a2, bg2, wh2, wfh2, depth, nei_t, 1000, bpe=et // 1000)
    ag2 = jnp.concatenate([ag_flat, et + ag_flat])
    gn2 = _gather_rows(hpk2, ag2, trim=False)
    wo2 = jnp.concatenate([p["bond_encoder"]["Wo"]["w"],
                           p["frag_encoder"]["Wo"]["w"]], axis=0)
    bo2 = jnp.concatenate([p["bond_encoder"]["Wo"]["b"],
                           p["frag_encoder"]["Wo"]["b"]])
    nh2 = _sumcat_dense(hx2, gn2, wo2, bo2, nei_t, "relu", True, 1000,
                        packed_g=True, bpe=nt // 1000)
    hbond, hinter = nh2[:nt], nh2[nt:]

    xc = jnp.concatenate([finput, hinter, hbond], axis=1)
    hnode_i = _dense(xc, p["W_c"]["w"], p["W_c"]["b"], "relu", False, 1000)
    hnode, hmess = _tree_encoder(p["inter_encoder"], hnode_i, f0, pos1h,
                                 ag_flat, bg_flat, depth, nei_t, 1000, 1000)

    agr = jnp.take(tree_agraph, roots, axis=0).reshape(-1)
    root_tab = jnp.concatenate([hnode_i, hmess], axis=0)
    root_idx = jnp.concatenate([roots, hnode_i.shape[0] + agr])
    rg = _gather_rows(root_tab, root_idx, ch=8)
    nr = roots.shape[0]
    fnode_r, g_r = rg[:nr], rg[nr:nr + agr.shape[0]]
    hroot = _sumcat_dense(fnode_r, g_r, p["W_root"]["w"], p["W_root"]["b"],
                          nei_t, "tanh", False, 128)

    return hroot, hnode, hinter, hbond, hatom


# speedup vs baseline: 2.4182x; 1.0013x over previous
import functools

import jax
import jax.numpy as jnp
from jax import lax
from jax.experimental import pallas as pl
from jax.experimental.pallas import tpu as pltpu
from jax.experimental.pallas import tpu_sc as plsc

_HID = 256
_NC = 2
_NS = 16
_NW = _NC * _NS


def _gather_rows(table, idx, ch=128, trim=True):
    n = idx.shape[0]
    d = table.shape[1]
    dt = table.dtype
    unit = _NW * ch
    n_pad = ((n + unit - 1) // unit) * unit
    if n_pad != n:
        idx = jnp.pad(idx, (0, n_pad - n))
    chunks = n_pad // unit

    pairs = chunks // 2
    tail = chunks % 2

    def body(tab_ref, idx_ref, out_ref, i0, i1, r0, r1, si0, si1, sg, sg1, sw0, sw1):
        wid = lax.axis_index("s") * _NC + lax.axis_index("c")
        base = wid * chunks

        def idx_cp(t, iv, sem):
            return pltpu.make_async_copy(idx_ref.at[pl.ds((base + t) * ch, ch)],
                                         iv, sem)

        def wb_cp(t, rv, sem):
            return pltpu.make_async_copy(rv, out_ref.at[pl.ds((base + t) * ch, ch)],
                                         sem)

        idx_cp(0, i0, si0).start()
        if chunks > 1:
            idx_cp(1, i1, si1).start()

        def g_cp(iv, rv, sem):
            return pltpu.make_async_copy(tab_ref.at[iv], rv, sem)

        def fire(t, iv, siv, rv, swv, sgv, p):
            idx_cp(t, iv, siv).wait()

            @pl.when(p > 0)
            def _():
                wb_cp(t, rv, swv).wait()

            g_cp(iv, rv, sgv).start()

        def drain(t, iv, siv, rv, swv, sgv):
            g_cp(iv, rv, sgv).wait()
            wb_cp(t, rv, swv).start()

            @pl.when(t + 2 < chunks)
            def _():
                idx_cp(t + 2, iv, siv).start()

        def pair(p, carry):
            t0 = jnp.int32(2 * p)
            t1 = t0 + 1
            fire(t0, i0, si0, r0, sw0, sg, p)
            fire(t1, i1, si1, r1, sw1, sg1, p)
            drain(t0, i0, si0, r0, sw0, sg)
            drain(t1, i1, si1, r1, sw1, sg1)
            return carry

        if pairs > 0:
            lax.fori_loop(0, pairs, pair, 0)
        if tail:
            t = jnp.int32(chunks - 1)
            fire(t, i0, si0, r0, sw0, sg, jnp.int32(pairs))
            drain(t, i0, si0, r0, sw0, sg)
        last = chunks - 1
        if chunks >= 2:
            prev = last - 1
            wb_cp(prev, r0 if prev % 2 == 0 else r1,
                  sw0 if prev % 2 == 0 else sw1).wait()
        wb_cp(last, r0 if last % 2 == 0 else r1,
              sw0 if last % 2 == 0 else sw1).wait()

    mesh = plsc.VectorSubcoreMesh(core_axis_name="c", subcore_axis_name="s")
    out = pl.kernel(
        body,
        mesh=mesh,
        out_type=jax.ShapeDtypeStruct((n_pad, d), dt),
        scratch_types=[
            pltpu.VMEM((ch,), jnp.int32),
            pltpu.VMEM((ch,), jnp.int32),
            pltpu.VMEM((ch, d), dt),
            pltpu.VMEM((ch, d), dt),
            pltpu.SemaphoreType.DMA,
            pltpu.SemaphoreType.DMA,
            pltpu.SemaphoreType.DMA,
            pltpu.SemaphoreType.DMA,
            pltpu.SemaphoreType.DMA,
            pltpu.SemaphoreType.DMA,
        ],
    )(table, idx)
    return out[:n] if trim else out


def _bdot(x, w):
    return jnp.dot(x.astype(jnp.bfloat16), w.astype(jnp.bfloat16),
                   preferred_element_type=jnp.float32)


def _apply_act(y, act):
    if act == "relu":
        return jnp.maximum(y, 0.0)
    if act == "tanh":
        return jnp.tanh(y)
    return y


def _mask0(y, i_blk, be, bpe):
    row = (i_blk % bpe) * be + lax.broadcasted_iota(jnp.int32, (be, 1), 0)
    return jnp.where(row == 0, 0.0, y)


def _dense(x, w, b, act, mask0, be, bpe=None):
    n, k = x.shape
    m = w.shape[1]
    if bpe is None:
        bpe = n // be
        w = w.reshape(1, k, m)
        b2 = b.reshape(1, 1, m)
    else:
        w = w.reshape(-1, k, m)
        b2 = b.reshape(-1, 1, m)

    def body(x_ref, w_ref, b_ref, o_ref):
        y = jnp.dot(x_ref[...], w_ref[0], preferred_element_type=jnp.float32)
        y = y + b_ref[0]
        y = _apply_act(y, act)
        if mask0:
            y = _mask0(y, pl.program_id(0), be, bpe)
        o_ref[...] = y

    return pl.pallas_call(
        body,
        grid=(n // be,),
        in_specs=[
            pl.BlockSpec((be, k), lambda i: (i, 0)),
            pl.BlockSpec((1, k, m), lambda i: (i // bpe, 0, 0)),
            pl.BlockSpec((1, 1, m), lambda i: (i // bpe, 0, 0)),
        ],
        out_specs=pl.BlockSpec((be, m), lambda i: (i, 0)),
        out_shape=jax.ShapeDtypeStruct((n, m), jnp.float32),
    )(x, w, b2)


def _pack_hc(h, c):
    hu = lax.bitcast_convert_type(h, jnp.uint32)
    cu = lax.bitcast_convert_type(c, jnp.uint32)
    pk = ((hu + 0x8000) & jnp.uint32(0xFFFF0000)) | ((cu + 0x8000) >> 16)
    return lax.bitcast_convert_type(pk, jnp.int32)


def _unpack_hc(g):
    gu = lax.bitcast_convert_type(g, jnp.uint32)
    hv = lax.bitcast_convert_type(gu & jnp.uint32(0xFFFF0000), jnp.float32)
    cv = lax.bitcast_convert_type(gu << 16, jnp.float32)
    return hv, cv


def _pack_cols(y):
    return _pack_hc(y[:, :_HID // 2], y[:, _HID // 2:])


def _unpack_cols(g):
    y1, y2 = _unpack_hc(g)
    return jnp.concatenate([y1, y2], axis=1)


def _lstm_init(a, be, bpe=None):
    e = a.shape[0]
    h = _HID
    if bpe is None:
        bpe = e // be

    def body(a_ref, h_ref, hc_ref):
        av = a_ref[...]
        gi = jax.nn.sigmoid(av[:, 0 * h:1 * h])
        go = jax.nn.sigmoid(av[:, 1 * h:2 * h])
        gg = jnp.tanh(av[:, 3 * h:4 * h])
        c_new = gi * gg
        h_new = go * jnp.tanh(c_new)
        h_new = _mask0(h_new, pl.program_id(0), be, bpe)
        c_new = _mask0(c_new, pl.program_id(0), be, bpe)
        h_ref[...] = h_new
        hc_ref[...] = _pack_hc(h_new, c_new)

    return pl.pallas_call(
        body,
        grid=(e // be,),
        in_specs=[pl.BlockSpec((be, 4 * h), lambda i: (i, 0))],
        out_specs=[
            pl.BlockSpec((be, h), lambda i: (i, 0)),
            pl.BlockSpec((be, h), lambda i: (i, 0)),
        ],
        out_shape=[
            jax.ShapeDtypeStruct((e, h), jnp.float32),
            jax.ShapeDtypeStruct((e, h), jnp.int32),
        ],
    )(a)


def _lstm_step(a, ghc, wh_all, wf_h, nk, be, last, bpe=None):
    e = a.shape[0]
    h = _HID
    if bpe is None:
        bpe = e // be
    wh_all = wh_all.reshape(-1, h, 3 * h)
    wf_h = wf_h.reshape(-1, h, h)

    def body(a_ref, g_ref, wh_ref, wf_ref, h_ref, p_ref):
        av = a_ref[...]
        ghv, gcv = _unpack_hc(g_ref[...])
        gcv = gcv.reshape(be, nk, h)
        hs = ghv.reshape(be, nk, h).sum(axis=1)
        zh = _bdot(hs, wh_ref[0])
        gi = jax.nn.sigmoid(av[:, 0 * h:1 * h] + zh[:, 0 * h:1 * h])
        go = jax.nn.sigmoid(av[:, 1 * h:2 * h] + zh[:, 1 * h:2 * h])
        gg = jnp.tanh(av[:, 3 * h:4 * h] + zh[:, 2 * h:3 * h])
        fpre = _bdot(ghv, wf_ref[0])
        gf = jax.nn.sigmoid(fpre.reshape(be, nk, h) + av[:, 2 * h:3 * h][:, None, :])
        c_new = gi * gg + (gf * gcv).sum(axis=1)
        h_new = go * jnp.tanh(c_new)
        h_new = _mask0(h_new, pl.program_id(0), be, bpe)
        c_new = _mask0(c_new, pl.program_id(0), be, bpe)
        h_ref[...] = h_new
        if last:
            p_ref[...] = _pack_cols(h_new)
        else:
            p_ref[...] = _pack_hc(h_new, c_new)

    pw = h // 2 if last else h
    return pl.pallas_call(
        body,
        grid=(e // be,),
        in_specs=[
            pl.BlockSpec((be, 4 * h), lambda i: (i, 0)),
            pl.BlockSpec((be * nk, h), lambda i: (i, 0)),
            pl.BlockSpec((1, h, 3 * h), lambda i: (i // bpe, 0, 0)),
            pl.BlockSpec((1, h, h), lambda i: (i // bpe, 0, 0)),
        ],
        out_specs=[
            pl.BlockSpec((be, h), lambda i: (i, 0)),
            pl.BlockSpec((be, pw), lambda i: (i, 0)),
        ],
        out_shape=[
            jax.ShapeDtypeStruct((e, h), jnp.float32),
            jax.ShapeDtypeStruct((e, pw), jnp.int32),
        ],
    )(a, ghc, wh_all, wf_h)


def _sumcat_dense(x1, g, w, b, nk, act, mask0, be, packed_g=False, emit_pk=False,
                  bpe=None):
    n, d1 = x1.shape
    h = _HID
    m = w.shape[1]
    if bpe is None:
        bpe = n // be
    w = w.reshape(-1, d1 + h, m)
    b2 = b.reshape(-1, 1, m)
    gw = h // 2 if packed_g else h

    def body(x_ref, g_ref, w_ref, b_ref, *o_refs):
        gv = g_ref[...]
        if packed_g:
            gv = _unpack_cols(gv)
        nei = gv.reshape(be, nk, h).sum(axis=1)
        xcat = jnp.concatenate([x_ref[...], nei], axis=1)
        y = _bdot(xcat, w_ref[0])
        y = y + b_ref[0]
        y = _apply_act(y, act)
        if mask0:
            y = _mask0(y, pl.program_id(0), be, bpe)
        o_refs[0][...] = y
        if emit_pk:
            o_refs[1][...] = _pack_cols(y)

    out_specs = [pl.BlockSpec((be, m), lambda i: (i, 0))]
    out_shape = [jax.ShapeDtypeStruct((n, m), jnp.float32)]
    if emit_pk:
        out_specs.append(pl.BlockSpec((be, m // 2), lambda i: (i, 0)))
        out_shape.append(jax.ShapeDtypeStruct((n, m // 2), jnp.int32))
    res = pl.pallas_call(
        body,
        grid=(n // be,),
        in_specs=[
            pl.BlockSpec((be, d1), lambda i: (i, 0)),
            pl.BlockSpec((be * nk, gw), lambda i: (i, 0)),
            pl.BlockSpec((1, d1 + h, m), lambda i: (i // bpe, 0, 0)),
            pl.BlockSpec((1, 1, m), lambda i: (i // bpe, 0, 0)),
        ],
        out_specs=out_specs,
        out_shape=out_shape,
    )(x1, g, w, b2)
    return res if emit_pk else res[0]


def _apre(hx_g, pos1h, w256, wpos, b, be, bpe, e):
    h = _HID
    m = w256.shape[-1]
    dp = pos1h.shape[1]
    w256 = w256.reshape(-1, h, m)
    wpos = wpos.reshape(-1, dp, m)
    b2 = b.reshape(-1, 1, m)

    def body(hx_ref, pos_ref, w_ref, wp_ref, b_ref, o_ref):
        y = jnp.dot(hx_ref[...], w_ref[0], preferred_element_type=jnp.float32)
        y = y + jnp.dot(pos_ref[...], wp_ref[0],
                        preferred_element_type=jnp.float32)
        o_ref[...] = y + b_ref[0]

    return pl.pallas_call(
        body,
        grid=(e // be,),
        in_specs=[
            pl.BlockSpec((be, h), lambda i: (i, 0)),
            pl.BlockSpec((be, dp), lambda i: (i, 0)),
            pl.BlockSpec((1, h, m), lambda i: (i // bpe, 0, 0)),
            pl.BlockSpec((1, dp, m), lambda i: (i // bpe, 0, 0)),
            pl.BlockSpec((1, 1, m), lambda i: (i // bpe, 0, 0)),
        ],
        out_specs=pl.BlockSpec((be, m), lambda i: (i, 0)),
        out_shape=jax.ShapeDtypeStruct((e, m), jnp.float32),
    )(hx_g, pos1h, w256, wpos, b2)


def _pack_lstm(lstm, in_sz, in_pad):
    wi, wo, wf, wg = lstm["Wi"], lstm["Wog"], lstm["Wf"], lstm["W"]
    w_all = jnp.concatenate(
        [wi["w"][:in_sz], wo["w"][:in_sz], wf["w"][:in_sz], wg["w"][:in_sz]], axis=1)
    if in_pad != in_sz:
        w_all = jnp.pad(w_all, ((0, in_pad - in_sz), (0, 0)))
    b_all = jnp.concatenate([wi["b"], wo["b"], wf["b"], wg["b"]])
    wh_all = jnp.concatenate(
        [wi["w"][in_sz:], wo["w"][in_sz:], wg["w"][in_sz:]], axis=1)
    wf_h = wf["w"][in_sz:]
    return w_all, b_all, wh_all, wf_h


def _run_lstm(a, bgraph_flat, wh_all, wf_h, depth, nk, be, bpe=None):
    h, hc = _lstm_init(a, be, bpe)
    for it in range(depth - 1):
        ghc = _gather_rows(hc, bgraph_flat, trim=False)
        h, hc = _lstm_step(a, ghc, wh_all, wf_h, nk, be, it == depth - 2, bpe)
    return h, hc


def _tree_encoder(enc, hx, f0, pos1h, agraph_flat, bgraph_flat, depth, nk, be_e, be_n):
    in_sz = _HID + pos1h.shape[1]
    e = f0.shape[0]
    w_all, b_all, wh_all, wf_h = _pack_lstm(enc["lstm"], in_sz, in_sz)
    hx_src = _gather_rows(hx, f0, trim=False)
    a = _apre(hx_src, pos1h, w_all[:_HID], w_all[_HID:], b_all,
              be_e, e // be_e, e)
    h, hpk = _run_lstm(a, bgraph_flat, wh_all, wf_h, depth, nk, be_e)
    gn = _gather_rows(hpk, agraph_flat, trim=False)
    nh = _sumcat_dense(hx, gn, enc["Wo"]["w"], enc["Wo"]["b"], nk, "relu", True,
                       be_n, packed_g=True)
    return nh, h


def kernel(tree_fnode, tree_fmess, tree_agraph, tree_bgraph, tree_cgraph,
           roots, graph_fnode, graph_fmess, graph_agraph, graph_bgraph, params):
    p = params
    depth = 3
    nei_g, nei_t, cw = 6, 6, 8

    genc = p["graph_encoder"]
    in_sz_g = 40 + 4 + 20
    w_all_g, b_all_g, wh_all_g, wf_h_g = _pack_lstm(genc["lstm"], in_sz_g, 128)
    src_atom = jnp.take(graph_fnode, graph_fmess[:, 0], axis=0)
    xg = jnp.concatenate(
        [jax.nn.one_hot(src_atom, 40, dtype=jnp.float32),
         jax.nn.one_hot(graph_fmess[:, 2], 4, dtype=jnp.float32),
         jax.nn.one_hot(graph_fmess[:, 3], 20, dtype=jnp.float32),
         jnp.zeros((graph_fmess.shape[0], 128 - in_sz_g), jnp.float32)], axis=1)
    a_g = _dense(xg, w_all_g, b_all_g, None, False, 1000)
    _, hpk_g = _run_lstm(a_g, graph_bgraph.reshape(-1), wh_all_g, wf_h_g,
                         depth, nei_g, 1000)
    gn_g = _gather_rows(hpk_g, graph_agraph.reshape(-1), trim=False)
    fnode_g = jnp.pad(jax.nn.one_hot(graph_fnode, 40, dtype=jnp.float32),
                      ((0, 0), (0, 88)))
    wo_g = jnp.concatenate(
        [jnp.pad(genc["Wo"]["w"][:40], ((0, 88), (0, 0))), genc["Wo"]["w"][40:]],
        axis=0)
    hatom, hatom_pk = _sumcat_dense(fnode_g, gn_g, wo_g, genc["Wo"]["b"], nei_g,
                                    "relu", True, 1000, packed_g=True,
                                    emit_pk=True)

    f0 = tree_fmess[:, 0]
    pos1h = jax.nn.one_hot(tree_fmess[:, 2], 20, dtype=jnp.float32)
    ag_flat = tree_agraph.reshape(-1)
    bg_flat = tree_bgraph.reshape(-1)

    emb_tab = jnp.concatenate([p["E_l"], p["E_i"], p["E_c"]], axis=0)
    nl, ni = p["E_l"].shape[0], p["E_i"].shape[0]
    emb_idx = jnp.concatenate(
        [tree_fnode[:, 2], nl + tree_fnode[:, 1], nl + ni + tree_fnode[:, 0]])
    nt = tree_fnode.shape[0]
    emb = _gather_rows(emb_tab, emb_idx)
    hnode_b, finput1, finput = emb[:nt], emb[nt:2 * nt], emb[2 * nt:3 * nt]

    g_cg = _gather_rows(hatom_pk, tree_cgraph.reshape(-1), trim=False)
    hn = _sumcat_dense(finput1, g_cg, p["W_i"]["w"], p["W_i"]["b"], cw,
                       "relu", False, 1000, packed_g=True)

    et = tree_fmess.shape[0]
    in_sz_t = _HID + 20
    pk_b = _pack_lstm(p["bond_encoder"]["lstm"], in_sz_t, in_sz_t)
    pk_f = _pack_lstm(p["frag_encoder"]["lstm"], in_sz_t, in_sz_t)
    w256_2 = jnp.concatenate([pk_b[0][:_HID], pk_f[0][:_HID]], axis=0)
    wpos_2 = jnp.concatenate([pk_b[0][_HID:], pk_f[0][_HID:]], axis=0)
    b_all2 = jnp.concatenate([pk_b[1], pk_f[1]])
    wh2 = jnp.concatenate([pk_b[2], pk_f[2]], axis=0)
    wfh2 = jnp.concatenate([pk_b[3], pk_f[3]], axis=0)
    hx2 = jnp.concatenate([hnode_b, hn], axis=0)
    f0_2 = jnp.concatenate([f0, nt + f0])
    hx_src2 = _gather_rows(hx2, f0_2, trim=False)
    pos2 = jnp.concatenate([pos1h, pos1h], axis=0)
    a2 = _apre(hx_src2, pos2, w256_2, wpos_2, b_all2, 1000, et // 1000, 2 * et)
    bg2 = jnp.concatenate([bg_flat, et + bg_flat])
    _, hpk2 = _run_lstm(a2, bg2, wh2, wfh2, depth, nei_t, 1000, bpe=et // 1000)
    ag2 = jnp.concatenate([ag_flat, et + ag_flat])
    gn2 = _gather_rows(hpk2, ag2, trim=False)
    wo2 = jnp.concatenate([p["bond_encoder"]["Wo"]["w"],
                           p["frag_encoder"]["Wo"]["w"]], axis=0)
    bo2 = jnp.concatenate([p["bond_encoder"]["Wo"]["b"],
                           p["frag_encoder"]["Wo"]["b"]])
    nh2 = _sumcat_dense(hx2, gn2, wo2, bo2, nei_t, "relu", True, 1000,
                        packed_g=True, bpe=nt // 1000)
    hbond, hinter = nh2[:nt], nh2[nt:]

    xc = jnp.concatenate([finput, hinter, hbond], axis=1)
    hnode_i = _dense(xc, p["W_c"]["w"], p["W_c"]["b"], "relu", False, 1000)
    hnode, hmess = _tree_encoder(p["inter_encoder"], hnode_i, f0, pos1h,
                                 ag_flat, bg_flat, depth, nei_t, 1000, 1000)

    agr = jnp.take(tree_agraph, roots, axis=0).reshape(-1)
    root_tab = jnp.concatenate([hnode_i, hmess], axis=0)
    root_idx = jnp.concatenate([roots, hnode_i.shape[0] + agr])
    rg = _gather_rows(root_tab, root_idx, ch=8)
    nr = roots.shape[0]
    fnode_r, g_r = rg[:nr], rg[nr:nr + agr.shape[0]]
    hroot = _sumcat_dense(fnode_r, g_r, p["W_root"]["w"], p["W_root"]["b"],
                          nei_t, "tanh", False, 128)

    return hroot, hnode, hinter, hbond, hatom


# speedup vs baseline: 2.4201x; 1.0007x over previous
import jax
import jax.numpy as jnp
from jax import lax
from jax.experimental import pallas as pl
from jax.experimental.pallas import tpu as pltpu
from jax.experimental.pallas import tpu_sc as plsc

_HID = 256
_NC = 2
_NS = 16
_NW = _NC * _NS


def _gather_rows(table, idx, ch=128, trim=True):
    n = idx.shape[0]
    d = table.shape[1]
    dt = table.dtype
    unit = _NW * ch
    n_pad = ((n + unit - 1) // unit) * unit
    if n_pad != n:
        idx = jnp.pad(idx, (0, n_pad - n))
    chunks = n_pad // unit

    pairs = chunks // 2
    tail = chunks % 2

    def body(tab_ref, idx_ref, out_ref, i0, i1, r0, r1, si0, si1, sg, sg1, sw0, sw1):
        wid = lax.axis_index("s") * _NC + lax.axis_index("c")
        base = wid * chunks

        def idx_cp(t, iv, sem):
            return pltpu.make_async_copy(idx_ref.at[pl.ds((base + t) * ch, ch)],
                                         iv, sem)

        def wb_cp(t, rv, sem):
            return pltpu.make_async_copy(rv, out_ref.at[pl.ds((base + t) * ch, ch)],
                                         sem)

        idx_cp(0, i0, si0).start()
        if chunks > 1:
            idx_cp(1, i1, si1).start()

        def g_cp(iv, rv, sem):
            return pltpu.make_async_copy(tab_ref.at[iv], rv, sem)

        def fire(t, iv, siv, rv, swv, sgv, p):
            idx_cp(t, iv, siv).wait()

            @pl.when(p > 0)
            def _():
                wb_cp(t, rv, swv).wait()

            g_cp(iv, rv, sgv).start()

        def drain(t, iv, siv, rv, swv, sgv):
            g_cp(iv, rv, sgv).wait()
            wb_cp(t, rv, swv).start()

            @pl.when(t + 2 < chunks)
            def _():
                idx_cp(t + 2, iv, siv).start()

        def pair(p, carry):
            t0 = jnp.int32(2 * p)
            t1 = t0 + 1
            fire(t0, i0, si0, r0, sw0, sg, p)
            fire(t1, i1, si1, r1, sw1, sg1, p)
            drain(t0, i0, si0, r0, sw0, sg)
            drain(t1, i1, si1, r1, sw1, sg1)
            return carry

        if pairs > 0:
            lax.fori_loop(0, pairs, pair, 0)
        if tail:
            t = jnp.int32(chunks - 1)
            fire(t, i0, si0, r0, sw0, sg, jnp.int32(pairs))
            drain(t, i0, si0, r0, sw0, sg)
        last = chunks - 1
        if chunks >= 2:
            prev = last - 1
            wb_cp(prev, r0 if prev % 2 == 0 else r1,
                  sw0 if prev % 2 == 0 else sw1).wait()
        wb_cp(last, r0 if last % 2 == 0 else r1,
              sw0 if last % 2 == 0 else sw1).wait()

    mesh = plsc.VectorSubcoreMesh(core_axis_name="c", subcore_axis_name="s")
    out = pl.kernel(
        body,
        mesh=mesh,
        out_type=jax.ShapeDtypeStruct((n_pad, d), dt),
        scratch_types=[
            pltpu.VMEM((ch,), jnp.int32),
            pltpu.VMEM((ch,), jnp.int32),
            pltpu.VMEM((ch, d), dt),
            pltpu.VMEM((ch, d), dt),
            pltpu.SemaphoreType.DMA,
            pltpu.SemaphoreType.DMA,
            pltpu.SemaphoreType.DMA,
            pltpu.SemaphoreType.DMA,
            pltpu.SemaphoreType.DMA,
            pltpu.SemaphoreType.DMA,
        ],
    )(table, idx)
    return out[:n] if trim else out


def _bdot(x, w):
    return jnp.dot(x.astype(jnp.bfloat16), w.astype(jnp.bfloat16),
                   preferred_element_type=jnp.float32)


def _apply_act(y, act):
    if act == "relu":
        return jnp.maximum(y, 0.0)
    if act == "tanh":
        return jnp.tanh(y)
    return y


def _mask0(y, i_blk, be, bpe):
    row = (i_blk % bpe) * be + lax.broadcasted_iota(jnp.int32, (be, 1), 0)
    return jnp.where(row == 0, 0.0, y)


def _dense(x, w, b, act, mask0, be, bpe=None):
    n, k = x.shape
    m = w.shape[1]
    if bpe is None:
        bpe = n // be
        w = w.reshape(1, k, m)
        b2 = b.reshape(1, 1, m)
    else:
        w = w.reshape(-1, k, m)
        b2 = b.reshape(-1, 1, m)

    def body(x_ref, w_ref, b_ref, o_ref):
        y = jnp.dot(x_ref[...], w_ref[0], preferred_element_type=jnp.float32)
        y = y + b_ref[0]
        y = _apply_act(y, act)
        if mask0:
            y = _mask0(y, pl.program_id(0), be, bpe)
        o_ref[...] = y

    return pl.pallas_call(
        body,
        grid=(n // be,),
        in_specs=[
            pl.BlockSpec((be, k), lambda i: (i, 0)),
            pl.BlockSpec((1, k, m), lambda i: (i // bpe, 0, 0)),
            pl.BlockSpec((1, 1, m), lambda i: (i // bpe, 0, 0)),
        ],
        out_specs=pl.BlockSpec((be, m), lambda i: (i, 0)),
        out_shape=jax.ShapeDtypeStruct((n, m), jnp.float32),
    )(x, w, b2)


def _pack_hc(h, c):
    hu = lax.bitcast_convert_type(h, jnp.uint32)
    cu = lax.bitcast_convert_type(c, jnp.uint32)
    pk = ((hu + 0x8000) & jnp.uint32(0xFFFF0000)) | ((cu + 0x8000) >> 16)
    return lax.bitcast_convert_type(pk, jnp.int32)


def _unpack_hc(g):
    gu = lax.bitcast_convert_type(g, jnp.uint32)
    hv = lax.bitcast_convert_type(gu & jnp.uint32(0xFFFF0000), jnp.float32)
    cv = lax.bitcast_convert_type(gu << 16, jnp.float32)
    return hv, cv


def _pack_cols(y):
    return _pack_hc(y[:, :_HID // 2], y[:, _HID // 2:])


def _unpack_cols(g):
    y1, y2 = _unpack_hc(g)
    return jnp.concatenate([y1, y2], axis=1)


def _lstm_init(a, be, bpe=None):
    e = a.shape[0]
    h = _HID
    if bpe is None:
        bpe = e // be

    def body(a_ref, h_ref, hc_ref):
        av = a_ref[...]
        gi = jax.nn.sigmoid(av[:, 0 * h:1 * h])
        go = jax.nn.sigmoid(av[:, 1 * h:2 * h])
        gg = jnp.tanh(av[:, 3 * h:4 * h])
        c_new = gi * gg
        h_new = go * jnp.tanh(c_new)
        h_new = _mask0(h_new, pl.program_id(0), be, bpe)
        c_new = _mask0(c_new, pl.program_id(0), be, bpe)
        h_ref[...] = h_new
        hc_ref[...] = _pack_hc(h_new, c_new)

    return pl.pallas_call(
        body,
        grid=(e // be,),
        in_specs=[pl.BlockSpec((be, 4 * h), lambda i: (i, 0))],
        out_specs=[
            pl.BlockSpec((be, h), lambda i: (i, 0)),
            pl.BlockSpec((be, h), lambda i: (i, 0)),
        ],
        out_shape=[
            jax.ShapeDtypeStruct((e, h), jnp.float32),
            jax.ShapeDtypeStruct((e, h), jnp.int32),
        ],
    )(a)


def _lstm_step(a, ghc, wh_all, wf_h, nk, be, last, bpe=None):
    e = a.shape[0]
    h = _HID
    if bpe is None:
        bpe = e // be
    wh_all = wh_all.reshape(-1, h, 3 * h)
    wf_h = wf_h.reshape(-1, h, h)

    def body(a_ref, g_ref, wh_ref, wf_ref, h_ref, p_ref):
        av = a_ref[...]
        ghv, gcv = _unpack_hc(g_ref[...])
        gcv = gcv.reshape(be, nk, h)
        hs = ghv.reshape(be, nk, h).sum(axis=1)
        zh = _bdot(hs, wh_ref[0])
        gi = jax.nn.sigmoid(av[:, 0 * h:1 * h] + zh[:, 0 * h:1 * h])
        go = jax.nn.sigmoid(av[:, 1 * h:2 * h] + zh[:, 1 * h:2 * h])
        gg = jnp.tanh(av[:, 3 * h:4 * h] + zh[:, 2 * h:3 * h])
        fpre = _bdot(ghv, wf_ref[0])
        gf = jax.nn.sigmoid(fpre.reshape(be, nk, h) + av[:, 2 * h:3 * h][:, None, :])
        c_new = gi * gg + (gf * gcv).sum(axis=1)
        h_new = go * jnp.tanh(c_new)
        h_new = _mask0(h_new, pl.program_id(0), be, bpe)
        c_new = _mask0(c_new, pl.program_id(0), be, bpe)
        h_ref[...] = h_new
        if last:
            p_ref[...] = _pack_cols(h_new)
        else:
            p_ref[...] = _pack_hc(h_new, c_new)

    pw = h // 2 if last else h
    return pl.pallas_call(
        body,
        grid=(e // be,),
        in_specs=[
            pl.BlockSpec((be, 4 * h), lambda i: (i, 0)),
            pl.BlockSpec((be * nk, h), lambda i: (i, 0)),
            pl.BlockSpec((1, h, 3 * h), lambda i: (i // bpe, 0, 0)),
            pl.BlockSpec((1, h, h), lambda i: (i // bpe, 0, 0)),
        ],
        out_specs=[
            pl.BlockSpec((be, h), lambda i: (i, 0)),
            pl.BlockSpec((be, pw), lambda i: (i, 0)),
        ],
        out_shape=[
            jax.ShapeDtypeStruct((e, h), jnp.float32),
            jax.ShapeDtypeStruct((e, pw), jnp.int32),
        ],
    )(a, ghc, wh_all, wf_h)


def _sumcat_dense(x1, g, w, b, nk, act, mask0, be, packed_g=False, emit_pk=False,
                  bpe=None):
    n, d1 = x1.shape
    h = _HID
    m = w.shape[1]
    if bpe is None:
        bpe = n // be
    w = w.reshape(-1, d1 + h, m)
    b2 = b.reshape(-1, 1, m)
    gw = h // 2 if packed_g else h

    def body(x_ref, g_ref, w_ref, b_ref, *o_refs):
        gv = g_ref[...]
        if packed_g:
            gv = _unpack_cols(gv)
        nei = gv.reshape(be, nk, h).sum(axis=1)
        xcat = jnp.concatenate([x_ref[...], nei], axis=1)
        y = _bdot(xcat, w_ref[0])
        y = y + b_ref[0]
        y = _apply_act(y, act)
        if mask0:
            y = _mask0(y, pl.program_id(0), be, bpe)
        o_refs[0][...] = y
        if emit_pk:
            o_refs[1][...] = _pack_cols(y)

    out_specs = [pl.BlockSpec((be, m), lambda i: (i, 0))]
    out_shape = [jax.ShapeDtypeStruct((n, m), jnp.float32)]
    if emit_pk:
        out_specs.append(pl.BlockSpec((be, m // 2), lambda i: (i, 0)))
        out_shape.append(jax.ShapeDtypeStruct((n, m // 2), jnp.int32))
    res = pl.pallas_call(
        body,
        grid=(n // be,),
        in_specs=[
            pl.BlockSpec((be, d1), lambda i: (i, 0)),
            pl.BlockSpec((be * nk, gw), lambda i: (i, 0)),
            pl.BlockSpec((1, d1 + h, m), lambda i: (i // bpe, 0, 0)),
            pl.BlockSpec((1, 1, m), lambda i: (i // bpe, 0, 0)),
        ],
        out_specs=out_specs,
        out_shape=out_shape,
    )(x1, g, w, b2)
    return res if emit_pk else res[0]


def _apre(hx_g, pos1h, w256, wpos, b, be, bpe, e):
    h = _HID
    m = w256.shape[-1]
    dp = pos1h.shape[1]
    w256 = w256.reshape(-1, h, m)
    wpos = wpos.reshape(-1, dp, m)
    b2 = b.reshape(-1, 1, m)

    def body(hx_ref, pos_ref, w_ref, wp_ref, b_ref, o_ref):
        y = jnp.dot(hx_ref[...], w_ref[0], preferred_element_type=jnp.float32)
        y = y + jnp.dot(pos_ref[...], wp_ref[0],
                        preferred_element_type=jnp.float32)
        o_ref[...] = y + b_ref[0]

    return pl.pallas_call(
        body,
        grid=(e // be,),
        in_specs=[
            pl.BlockSpec((be, h), lambda i: (i, 0)),
            pl.BlockSpec((be, dp), lambda i: (i, 0)),
            pl.BlockSpec((1, h, m), lambda i: (i // bpe, 0, 0)),
            pl.BlockSpec((1, dp, m), lambda i: (i // bpe, 0, 0)),
            pl.BlockSpec((1, 1, m), lambda i: (i // bpe, 0, 0)),
        ],
        out_specs=pl.BlockSpec((be, m), lambda i: (i, 0)),
        out_shape=jax.ShapeDtypeStruct((e, m), jnp.float32),
    )(hx_g, pos1h, w256, wpos, b2)


def _pack_lstm(lstm, in_sz, in_pad):
    wi, wo, wf, wg = lstm["Wi"], lstm["Wog"], lstm["Wf"], lstm["W"]
    w_all = jnp.concatenate(
        [wi["w"][:in_sz], wo["w"][:in_sz], wf["w"][:in_sz], wg["w"][:in_sz]], axis=1)
    if in_pad != in_sz:
        w_all = jnp.pad(w_all, ((0, in_pad - in_sz), (0, 0)))
    b_all = jnp.concatenate([wi["b"], wo["b"], wf["b"], wg["b"]])
    wh_all = jnp.concatenate(
        [wi["w"][in_sz:], wo["w"][in_sz:], wg["w"][in_sz:]], axis=1)
    wf_h = wf["w"][in_sz:]
    return w_all, b_all, wh_all, wf_h


def _run_lstm(a, bgraph_flat, wh_all, wf_h, depth, nk, be, bpe=None):
    h, hc = _lstm_init(a, be, bpe)
    for it in range(depth - 1):
        ghc = _gather_rows(hc, bgraph_flat, trim=False)
        h, hc = _lstm_step(a, ghc, wh_all, wf_h, nk, be, it == depth - 2, bpe)
    return h, hc


def _tree_encoder(enc, hx, f0, pos1h, agraph_flat, bgraph_flat, depth, nk, be_e, be_n):
    in_sz = _HID + pos1h.shape[1]
    e = f0.shape[0]
    w_all, b_all, wh_all, wf_h = _pack_lstm(enc["lstm"], in_sz, in_sz)
    hx_src = _gather_rows(hx, f0, trim=False)
    a = _apre(hx_src, pos1h, w_all[:_HID], w_all[_HID:], b_all,
              be_e, e // be_e, e)
    h, hpk = _run_lstm(a, bgraph_flat, wh_all, wf_h, depth, nk, be_e)
    gn = _gather_rows(hpk, agraph_flat, trim=False)
    nh = _sumcat_dense(hx, gn, enc["Wo"]["w"], enc["Wo"]["b"], nk, "relu", True,
                       be_n, packed_g=True)
    return nh, h


def kernel(tree_fnode, tree_fmess, tree_agraph, tree_bgraph, tree_cgraph,
           roots, graph_fnode, graph_fmess, graph_agraph, graph_bgraph, params):
    p = params
    depth = 3
    nei_g, nei_t, cw = 6, 6, 8

    genc = p["graph_encoder"]
    in_sz_g = 40 + 4 + 20
    w_all_g, b_all_g, wh_all_g, wf_h_g = _pack_lstm(genc["lstm"], in_sz_g, 128)
    src_atom = jnp.take(graph_fnode, graph_fmess[:, 0], axis=0)
    xg = jnp.concatenate(
        [jax.nn.one_hot(src_atom, 40, dtype=jnp.float32),
         jax.nn.one_hot(graph_fmess[:, 2], 4, dtype=jnp.float32),
         jax.nn.one_hot(graph_fmess[:, 3], 20, dtype=jnp.float32),
         jnp.zeros((graph_fmess.shape[0], 128 - in_sz_g), jnp.float32)], axis=1)
    a_g = _dense(xg, w_all_g, b_all_g, None, False, 1000)
    _, hpk_g = _run_lstm(a_g, graph_bgraph.reshape(-1), wh_all_g, wf_h_g,
                         depth, nei_g, 1000)
    gn_g = _gather_rows(hpk_g, graph_agraph.reshape(-1), trim=False)
    fnode_g = jnp.pad(jax.nn.one_hot(graph_fnode, 40, dtype=jnp.float32),
                      ((0, 0), (0, 88)))
    wo_g = jnp.concatenate(
        [jnp.pad(genc["Wo"]["w"][:40], ((0, 88), (0, 0))), genc["Wo"]["w"][40:]],
        axis=0)
    hatom, hatom_pk = _sumcat_dense(fnode_g, gn_g, wo_g, genc["Wo"]["b"], nei_g,
                                    "relu", True, 1000, packed_g=True,
                                    emit_pk=True)

    f0 = tree_fmess[:, 0]
    pos1h = jax.nn.one_hot(tree_fmess[:, 2], 20, dtype=jnp.float32)
    ag_flat = tree_agraph.reshape(-1)
    bg_flat = tree_bgraph.reshape(-1)

    emb_tab = jnp.concatenate([p["E_l"], p["E_i"], p["E_c"]], axis=0)
    nl, ni = p["E_l"].shape[0], p["E_i"].shape[0]
    emb_idx = jnp.concatenate(
        [tree_fnode[:, 2], nl + tree_fnode[:, 1], nl + ni + tree_fnode[:, 0]])
    nt = tree_fnode.shape[0]
    emb = _gather_rows(emb_tab, emb_idx)
    hnode_b, finput1, finput = emb[:nt], emb[nt:2 * nt], emb[2 * nt:3 * nt]

    g_cg = _gather_rows(hatom_pk, tree_cgraph.reshape(-1), trim=False)
    hn = _sumcat_dense(finput1, g_cg, p["W_i"]["w"], p["W_i"]["b"], cw,
                       "relu", False, 1000, packed_g=True)

    et = tree_fmess.shape[0]
    in_sz_t = _HID + 20
    pk_b = _pack_lstm(p["bond_encoder"]["lstm"], in_sz_t, in_sz_t)
    pk_f = _pack_lstm(p["frag_encoder"]["lstm"], in_sz_t, in_sz_t)
    w256_2 = jnp.concatenate([pk_b[0][:_HID], pk_f[0][:_HID]], axis=0)
    wpos_2 = jnp.concatenate([pk_b[0][_HID:], pk_f[0][_HID:]], axis=0)
    b_all2 = jnp.concatenate([pk_b[1], pk_f[1]])
    wh2 = jnp.concatenate([pk_b[2], pk_f[2]], axis=0)
    wfh2 = jnp.concatenate([pk_b[3], pk_f[3]], axis=0)
    hx2 = jnp.concatenate([hnode_b, hn], axis=0)
    f0_2 = jnp.concatenate([f0, nt + f0])
    hx_src2 = _gather_rows(hx2, f0_2, trim=False)
    pos2 = jnp.concatenate([pos1h, pos1h], axis=0)
    a2 = _apre(hx_src2, pos2, w256_2, wpos_2, b_all2, 1000, et // 1000, 2 * et)
    bg2 = jnp.concatenate([bg_flat, et + bg_flat])
    _, hpk2 = _run_lstm(a2, bg2, wh2, wfh2, depth, nei_t, 1000, bpe=et // 1000)
    ag2 = jnp.concatenate([ag_flat, et + ag_flat])
    gn2 = _gather_rows(hpk2, ag2, trim=False)
    wo2 = jnp.concatenate([p["bond_encoder"]["Wo"]["w"],
                           p["frag_encoder"]["Wo"]["w"]], axis=0)
    bo2 = jnp.concatenate([p["bond_encoder"]["Wo"]["b"],
                           p["frag_encoder"]["Wo"]["b"]])
    nh2 = _sumcat_dense(hx2, gn2, wo2, bo2, nei_t, "relu", True, 1000,
                        packed_g=True, bpe=nt // 1000)
    hbond, hinter = nh2[:nt], nh2[nt:]

    xc = jnp.concatenate([finput, hinter, hbond], axis=1)
    hnode_i = _dense(xc, p["W_c"]["w"], p["W_c"]["b"], "relu", False, 1000)
    hnode, hmess = _tree_encoder(p["inter_encoder"], hnode_i, f0, pos1h,
                                 ag_flat, bg_flat, depth, nei_t, 1000, 1000)

    agr = jnp.take(tree_agraph, roots, axis=0).reshape(-1)
    root_tab = jnp.concatenate([hnode_i, hmess], axis=0)
    root_idx = jnp.concatenate([roots, hnode_i.shape[0] + agr])
    rg = _gather_rows(root_tab, root_idx, ch=8)
    nr = roots.shape[0]
    fnode_r, g_r = rg[:nr], rg[nr:nr + agr.shape[0]]
    hroot = _sumcat_dense(fnode_r, g_r, p["W_root"]["w"], p["W_root"]["b"],
                          nei_t, "tanh", False, 128)

    return hroot, hnode, hinter, hbond, hatom
